# baseline (speedup 1.0000x reference)
import functools

import jax
import jax.numpy as jnp
from jax import lax
from jax.experimental import pallas as pl
from jax.experimental.pallas import tpu as pltpu
from jax.experimental.pallas import tpu_sc as plsc

N = 4096
D = 128
NGRP = N // 16
NTILES = 32
GPT = NGRP // NTILES
WPT = GPT * N
CAPW = 8192
CAPE = 4096
CHUNK = 128
HALF = N // 2
DUMMY = HALF

BLK1 = 256


def _pack_body(thr_ref, a_ref, q_ref, pk_ref, dout_ref, din_ref):
    i = pl.program_id(0)
    mask = (a_ref[...] >= thr_ref[0]).astype(jnp.float32)
    dout_ref[...] = jnp.sum(mask, axis=1)[None, :]

    @pl.when(i == 0)
    def _():
        din_ref[...] = jnp.zeros_like(din_ref)

    din_ref[...] += jnp.sum(mask, axis=0)[None, :]
    pk = lax.dot_general(q_ref[...], mask.astype(jnp.bfloat16),
                         (((1,), (1,)), ((), ())),
                         preferred_element_type=jnp.float32)
    pk_ref[...] = pk.astype(jnp.int32)


def _pack_and_degrees(A, thr):
    col = lax.iota(jnp.int32, N)
    q = jnp.where((col[None, :] // 16) == lax.iota(jnp.int32, NGRP)[:, None],
                  jnp.exp2((col % 16).astype(jnp.float32))[None, :],
                  0.0).astype(jnp.bfloat16)
    grid = (N // BLK1,)
    return pl.pallas_call(
        _pack_body,
        grid=grid,
        in_specs=[
            pl.BlockSpec(memory_space=pltpu.SMEM),
            pl.BlockSpec((BLK1, N), lambda i: (i, 0)),
            pl.BlockSpec((NGRP, N), lambda i: (0, 0)),
        ],
        out_specs=[
            pl.BlockSpec((NGRP, BLK1), lambda i: (0, i)),
            pl.BlockSpec((1, BLK1), lambda i: (0, i)),
            pl.BlockSpec((1, N), lambda i: (0, 0)),
        ],
        out_shape=[
            jax.ShapeDtypeStruct((NGRP, N), jnp.int32),
            jax.ShapeDtypeStruct((1, N), jnp.float32),
            jax.ShapeDtypeStruct((1, N), jnp.float32),
        ],
        compiler_params=pltpu.CompilerParams(
            dimension_semantics=("arbitrary",),
        ),
    )(thr, A, q)


def _extract_body(pk_hbm, esrc_hbm, edst_hbm, cnt_hbm,
                  pk_v, src_v, dst_v, nzw_v, nzi_v, out16_v):
    c = lax.axis_index("c")
    s = lax.axis_index("s")
    wid = c * 16 + s
    g0 = wid * GPT
    iota = lax.iota(jnp.int32, 16)

    pltpu.sync_copy(pk_hbm.at[pl.ds(g0, GPT), :], pk_v)

    off = jnp.int32(0)
    for gl in range(GPT):
        def body_a(i, off, gl=gl):
            v = pk_v[gl, pl.ds(i * 16, 16)]
            mi = (v != 0).astype(jnp.int32)
            excl = plsc.cumsum(mi) - mi
            dest = jnp.where(mi != 0, off + excl, CAPW)
            plsc.store_scatter(nzw_v, [dest], v)
            plsc.store_scatter(nzi_v, [dest], gl * N + i * 16 + iota)
            return jnp.minimum(off + jnp.sum(mi), CAPW - 16)
        off = lax.fori_loop(0, N // 16, body_a, off)

    nw = off
    dbase0 = (g0 - c * (HALF // 16)) * 16

    def body_b(j, ec):
        lanes_left = nw - j * 16
        lanem = iota < lanes_left
        w = jnp.where(lanem, nzw_v[pl.ds(j * 16, 16)], 0)
        fi = nzi_v[pl.ds(j * 16, 16)]
        r = fi & (N - 1)
        gl = lax.shift_right_logical(fi, 12)
        dstbase = dbase0 + gl * 16

        def peel_cond(carry):
            w, _ = carry
            return jnp.max(w, axis=0) > 0

        def peel(carry):
            w, ec = carry
            b = w & (-w)
            mi = (b != 0).astype(jnp.int32)
            bf = b.astype(jnp.float32)
            bi = lax.shift_right_logical(
                lax.bitcast_convert_type(bf, jnp.int32), 23) - 127
            excl = plsc.cumsum(mi) - mi
            dest = jnp.where(mi != 0, ec + excl, CAPE)
            plsc.store_scatter(src_v, [dest], r)
            plsc.store_scatter(dst_v, [dest], dstbase + bi)
            ec = jnp.minimum(ec + jnp.sum(mi), CAPE - 16)
            return (w & (w - 1), ec)

        _, ec = lax.while_loop(peel_cond, peel, (w, ec))
        return ec

    ec = lax.fori_loop(0, (nw + 15) // 16, body_b, jnp.int32(0))

    target = ((ec + CHUNK - 1) // CHUNK) * CHUNK

    def pad_cond(o):
        return o < target

    def pad(o):
        src_v[pl.ds(o, 16)] = jnp.zeros((16,), jnp.int32)
        dst_v[pl.ds(o, 16)] = jnp.full((16,), DUMMY, jnp.int32)
        return o + 16

    lax.while_loop(pad_cond, pad, ec)

    out16_v[...] = jnp.full((16,), ec, jnp.int32)
    pltpu.sync_copy(out16_v, cnt_hbm.at[pl.ds(wid * 16, 16)])
    pltpu.sync_copy(src_v.at[pl.ds(0, CAPE)], esrc_hbm.at[wid])
    pltpu.sync_copy(dst_v.at[pl.ds(0, CAPE)], edst_hbm.at[wid])


def _extract(packed):
    mesh = plsc.VectorSubcoreMesh(core_axis_name="c", subcore_axis_name="s")
    return pl.kernel(
        _extract_body,
        out_type=[
            jax.ShapeDtypeStruct((NTILES, CAPE), jnp.int32),
            jax.ShapeDtypeStruct((NTILES, CAPE), jnp.int32),
            jax.ShapeDtypeStruct((NTILES * 16,), jnp.int32),
        ],
        mesh=mesh,
        scratch_types=[
            pltpu.VMEM((GPT, N), jnp.int32),
            pltpu.VMEM((CAPE + 16,), jnp.int32),
            pltpu.VMEM((CAPE + 16,), jnp.int32),
            pltpu.VMEM((CAPW + 16,), jnp.int32),
            pltpu.VMEM((CAPW + 16,), jnp.int32),
            pltpu.VMEM((16,), jnp.int32),
        ],
        compiler_params=pltpu.CompilerParams(needs_layout_passes=False),
    )(packed)


def _agg_body(hs_hbm, esrc_hbm, edst_hbm, cnt_hbm, out_hbm,
              sidx_v, didx_v, rows_v, cnt_v, shared, sem):
    c = lax.axis_index("c")
    s = lax.axis_index("s")
    wid = c * 16 + s
    stripe = s * 128

    pltpu.sync_copy(hs_hbm.at[pl.ds(c * HALF + stripe, 128), :],
                    shared.at[pl.ds(stripe, 128), :])
    plsc.subcore_barrier()

    pltpu.sync_copy(cnt_hbm.at[pl.ds(wid * 16, 16)], cnt_v)
    n = jnp.max(cnt_v[...], axis=0)

    def chunk(ci, _):
        pltpu.sync_copy(esrc_hbm.at[wid, pl.ds(ci * CHUNK, CHUNK)], sidx_v)
        pltpu.sync_copy(edst_hbm.at[wid, pl.ds(ci * CHUNK, CHUNK)], didx_v)
        pltpu.async_copy(hs_hbm.at[sidx_v], rows_v, sem).wait()
        pltpu.sync_copy(rows_v, shared.at[didx_v], add=True)
        return 0

    lax.fori_loop(0, (n + CHUNK - 1) // CHUNK, chunk, 0)
    plsc.subcore_barrier()
    pltpu.sync_copy(shared.at[pl.ds(stripe, 128), :],
                    out_hbm.at[pl.ds(wid * 128, 128), :])


def _aggregate(hs, esrc, edst, cnt):
    mesh = plsc.VectorSubcoreMesh(core_axis_name="c", subcore_axis_name="s")
    return pl.kernel(
        _agg_body,
        out_type=jax.ShapeDtypeStruct((N, D), jnp.float32),
        mesh=mesh,
        scratch_types=[
            pltpu.VMEM((CHUNK,), jnp.int32),
            pltpu.VMEM((CHUNK,), jnp.int32),
            pltpu.VMEM((CHUNK, D), jnp.float32),
            pltpu.VMEM((16,), jnp.int32),
            pltpu.VMEM_SHARED((HALF + 16, D), jnp.float32),
            pltpu.SemaphoreType.DMA,
        ],
        compiler_params=pltpu.CompilerParams(needs_layout_passes=False),
    )(hs, esrc, edst, cnt)


def _prep_body(f_ref, no_ref, out_ref):
    out_ref[...] = f_ref[...] * no_ref[...]


def _prep(features, no_col):
    grid = (8,)
    blk = N // 8
    return pl.pallas_call(
        _prep_body,
        grid=grid,
        in_specs=[
            pl.BlockSpec((blk, D), lambda i: (i, 0)),
            pl.BlockSpec((blk, 1), lambda i: (i, 0)),
        ],
        out_specs=pl.BlockSpec((blk, D), lambda i: (i, 0)),
        out_shape=jax.ShapeDtypeStruct((N, D), jnp.float32),
    )(features, no_col)


def _post_body(agg_ref, ni_ref, no_ref, w_ref, b_ref, h_ref, hs_ref):
    hd = agg_ref[...] * ni_ref[...]
    h = jax.nn.relu(
        jnp.dot(hd, w_ref[...], preferred_element_type=jnp.float32)
        + b_ref[...])
    h_ref[...] = h
    hs_ref[...] = h * no_ref[...]


def _post(agg, ni_col, no_col, W, b):
    grid = (8,)
    blk = N // 8
    return pl.pallas_call(
        _post_body,
        grid=grid,
        in_specs=[
            pl.BlockSpec((blk, D), lambda i: (i, 0)),
            pl.BlockSpec((blk, 1), lambda i: (i, 0)),
            pl.BlockSpec((blk, 1), lambda i: (i, 0)),
            pl.BlockSpec((D, D), lambda i: (0, 0)),
            pl.BlockSpec((1, D), lambda i: (0, 0)),
        ],
        out_specs=[
            pl.BlockSpec((blk, D), lambda i: (i, 0)),
            pl.BlockSpec((blk, D), lambda i: (i, 0)),
        ],
        out_shape=[
            jax.ShapeDtypeStruct((N, D), jnp.float32),
            jax.ShapeDtypeStruct((N, D), jnp.float32),
        ],
    )(agg, ni_col, no_col, W, b[None, :])


def kernel(A, features, threshold, W1, b1, W2, b2):
    thr = jnp.asarray(threshold, jnp.float32).reshape(1)
    packed, dout, din = _pack_and_degrees(A, thr)
    no_col = lax.rsqrt(dout[0] + 1.0)[:, None]
    ni_col = lax.rsqrt(din[0] + 1.0)[:, None]
    esrc, edst, cnt = _extract(packed)
    hs0 = _prep(features, no_col)
    agg1 = _aggregate(hs0, esrc, edst, cnt)
    h1, hs1 = _post(agg1, ni_col, no_col, W1, b1)
    agg2 = _aggregate(hs1, esrc, edst, cnt)
    h2, _ = _post(agg2, ni_col, no_col, W2, b2)
    return (h1, h2)


# speedup vs baseline: 1.1211x; 1.1211x over previous
import jax
import jax.numpy as jnp
from jax import lax
from jax.experimental import pallas as pl
from jax.experimental.pallas import tpu as pltpu
from jax.experimental.pallas import tpu_sc as plsc

N = 4096
D = 128
NG32 = N // 32
NTILES = 32
GPT = NG32 // NTILES
CAPW = 4096
CAPE = 4096
CHUNK = 128
HALF = N // 2
DUMMY = HALF

BLK1 = 256


def _pack_body(thr_ref, a_ref, q_ref, pk_ref, dout_ref, din_ref):
    i = pl.program_id(0)
    mask = (a_ref[...] >= thr_ref[0]).astype(jnp.float32)
    dout_ref[...] = jnp.sum(mask, axis=1)[None, :]

    @pl.when(i == 0)
    def _():
        din_ref[...] = jnp.zeros_like(din_ref)

    din_ref[...] += jnp.sum(mask, axis=0)[None, :]
    pk = lax.dot_general(q_ref[...], mask.astype(jnp.bfloat16),
                         (((1,), (1,)), ((), ())),
                         preferred_element_type=jnp.float32).astype(jnp.int32)
    lo = lax.slice(pk, (0, 0), (NG32, BLK1))
    hi = lax.slice(pk, (NG32, 0), (2 * NG32, BLK1))
    pk_ref[...] = lo | lax.shift_left(hi, 16)


def _pack_and_degrees(A, thr):
    col = lax.iota(jnp.int32, N)
    gp = lax.iota(jnp.int32, 2 * NG32)[:, None]
    grp16 = jnp.where(gp < NG32, 2 * gp, 2 * (gp - NG32) + 1)
    q = jnp.where((col[None, :] // 16) == grp16,
                  jnp.exp2((col % 16).astype(jnp.float32))[None, :],
                  0.0).astype(jnp.bfloat16)
    grid = (N // BLK1,)
    return pl.pallas_call(
        _pack_body,
        grid=grid,
        in_specs=[
            pl.BlockSpec(memory_space=pltpu.SMEM),
            pl.BlockSpec((BLK1, N), lambda i: (i, 0)),
            pl.BlockSpec((2 * NG32, N), lambda i: (0, 0)),
        ],
        out_specs=[
            pl.BlockSpec((NG32, BLK1), lambda i: (0, i)),
            pl.BlockSpec((1, BLK1), lambda i: (0, i)),
            pl.BlockSpec((1, N), lambda i: (0, 0)),
        ],
        out_shape=[
            jax.ShapeDtypeStruct((NG32, N), jnp.int32),
            jax.ShapeDtypeStruct((1, N), jnp.float32),
            jax.ShapeDtypeStruct((1, N), jnp.float32),
        ],
        compiler_params=pltpu.CompilerParams(
            dimension_semantics=("arbitrary",),
        ),
    )(thr, A, q)


def _agg_loop(hs_hbm, src_v, dst_v, n, shared,
              sidx0, didx0, rows0, sem0, sidx1, didx1, rows1, sem1):
    nch = (n + CHUNK - 1) // CHUNK

    def load_idx(ci, sref, dref):
        for k in range(CHUNK // 16):
            sref[pl.ds(k * 16, 16)] = src_v[pl.ds(ci * CHUNK + k * 16, 16)]
            dref[pl.ds(k * 16, 16)] = dst_v[pl.ds(ci * CHUNK + k * 16, 16)]

    @pl.when(nch > 0)
    def _():
        load_idx(0, sidx0, didx0)
        pltpu.async_copy(hs_hbm.at[sidx0], rows0, sem0)

    def body(ci, _):
        p = lax.rem(ci, 2)

        @pl.when((p == 0) & (ci + 1 < nch))
        def _():
            load_idx(ci + 1, sidx1, didx1)
            pltpu.async_copy(hs_hbm.at[sidx1], rows1, sem1)

        @pl.when((p == 1) & (ci + 1 < nch))
        def _():
            load_idx(ci + 1, sidx0, didx0)
            pltpu.async_copy(hs_hbm.at[sidx0], rows0, sem0)

        @pl.when(p == 0)
        def _():
            pltpu.make_async_copy(hs_hbm.at[pl.ds(0, CHUNK), :], rows0,
                                  sem0).wait()
            pltpu.sync_copy(rows0, shared.at[didx0], add=True)

        @pl.when(p == 1)
        def _():
            pltpu.make_async_copy(hs_hbm.at[pl.ds(0, CHUNK), :], rows1,
                                  sem1).wait()
            pltpu.sync_copy(rows1, shared.at[didx1], add=True)

        return 0

    lax.fori_loop(0, nch, body, 0)


def _extract_agg_body(pk_hbm, hs_hbm, esrc_hbm, edst_hbm, cnt_hbm, agg_hbm,
                      pk_v, src_v, dst_v, nzw_v, nzi_v, out16_v,
                      sidx0, didx0, rows0, sidx1, didx1, rows1,
                      shared, sem0, sem1):
    c = lax.axis_index("c")
    s = lax.axis_index("s")
    wid = c * 16 + s
    g0 = wid * GPT
    iota = lax.iota(jnp.int32, 16)

    pltpu.sync_copy(pk_hbm.at[pl.ds(g0, GPT), :], pk_v)
    pltpu.sync_copy(hs_hbm.at[pl.ds(wid * 128, 128), :],
                    shared.at[pl.ds(s * 128, 128), :])

    offv = jnp.zeros((16,), jnp.int32)
    for gl in range(GPT):
        def body_a(i, offv, gl=gl):
            v = pk_v[gl, pl.ds(i * 16, 16)]
            mm = v != 0
            mi = mm.astype(jnp.int32)
            excl = plsc.cumsum(mi) - mi
            dest = jnp.where(mm, offv + excl, CAPW)
            plsc.store_scatter(nzw_v, [dest], v)
            plsc.store_scatter(nzi_v, [dest], gl * N + i * 16 + iota)
            cnt = plsc.all_reduce_population_count(mm)
            return jnp.minimum(offv + cnt, CAPW - 16)
        offv = lax.fori_loop(0, N // 16, body_a, offv)

    nw = jnp.max(offv, axis=0)
    dbase0 = g0 * 32 - c * HALF

    def body_b(j, ecv):
        lanem = iota < (nw - j * 16)
        w = jnp.where(lanem, nzw_v[pl.ds(j * 16, 16)], 0)
        fi = nzi_v[pl.ds(j * 16, 16)]
        r = fi & (N - 1)
        gl = lax.shift_right_logical(fi, 12)
        dstbase = dbase0 + gl * 32

        def peel_cond(carry):
            return jnp.any(carry[0] != 0)

        def peel(carry):
            w, ecv = carry
            b = w & (-w)
            mm = b != 0
            bf = b.astype(jnp.float32)
            bi = (lax.shift_right_logical(
                lax.bitcast_convert_type(bf, jnp.int32), 23) & 0xFF) - 127
            mi = mm.astype(jnp.int32)
            excl = plsc.cumsum(mi) - mi
            dest = jnp.where(mm, ecv + excl, CAPE)
            plsc.store_scatter(src_v, [dest], r)
            plsc.store_scatter(dst_v, [dest], dstbase + bi)
            cnt = plsc.all_reduce_population_count(mm)
            ecv = jnp.minimum(ecv + cnt, CAPE - 16)
            return (w & (w - 1), ecv)

        _, ecv = lax.while_loop(peel_cond, peel, (w, ecv))
        return ecv

    ecv = lax.fori_loop(0, (nw + 15) // 16, body_b,
                        jnp.zeros((16,), jnp.int32))
    ec = jnp.max(ecv, axis=0)

    target = ((ec + CHUNK - 1) // CHUNK) * CHUNK

    def pad(o):
        src_v[pl.ds(o, 16)] = jnp.zeros((16,), jnp.int32)
        dst_v[pl.ds(o, 16)] = jnp.full((16,), DUMMY, jnp.int32)
        return o + 16

    lax.while_loop(lambda o: o < target, pad, ec)

    out16_v[...] = jnp.full((16,), ec, jnp.int32)
    pltpu.sync_copy(out16_v, cnt_hbm.at[pl.ds(wid * 16, 16)])
    pltpu.sync_copy(src_v.at[pl.ds(0, CAPE)], esrc_hbm.at[wid])
    pltpu.sync_copy(dst_v.at[pl.ds(0, CAPE)], edst_hbm.at[wid])

    _agg_loop(hs_hbm, src_v, dst_v, ec, shared,
              sidx0, didx0, rows0, sem0, sidx1, didx1, rows1, sem1)
    pltpu.sync_copy(shared.at[pl.ds(s * 128, 128), :],
                    agg_hbm.at[pl.ds(wid * 128, 128), :])


def _extract_agg(packed, hs0):
    mesh = plsc.VectorSubcoreMesh(core_axis_name="c", subcore_axis_name="s")
    return pl.kernel(
        _extract_agg_body,
        out_type=[
            jax.ShapeDtypeStruct((NTILES, CAPE), jnp.int32),
            jax.ShapeDtypeStruct((NTILES, CAPE), jnp.int32),
            jax.ShapeDtypeStruct((NTILES * 16,), jnp.int32),
            jax.ShapeDtypeStruct((N, D), jnp.float32),
        ],
        mesh=mesh,
        scratch_types=[
            pltpu.VMEM((GPT, N), jnp.int32),
            pltpu.VMEM((CAPE + 16,), jnp.int32),
            pltpu.VMEM((CAPE + 16,), jnp.int32),
            pltpu.VMEM((CAPW + 16,), jnp.int32),
            pltpu.VMEM((CAPW + 16,), jnp.int32),
            pltpu.VMEM((16,), jnp.int32),
            pltpu.VMEM((CHUNK,), jnp.int32),
            pltpu.VMEM((CHUNK,), jnp.int32),
            pltpu.VMEM((CHUNK, D), jnp.float32),
            pltpu.VMEM((CHUNK,), jnp.int32),
            pltpu.VMEM((CHUNK,), jnp.int32),
            pltpu.VMEM((CHUNK, D), jnp.float32),
            pltpu.VMEM_SHARED((HALF + 16, D), jnp.float32),
            pltpu.SemaphoreType.DMA,
            pltpu.SemaphoreType.DMA,
        ],
        compiler_params=pltpu.CompilerParams(needs_layout_passes=False),
    )(packed, hs0)


def _agg2_body(hs_hbm, esrc_hbm, edst_hbm, cnt_hbm, agg_hbm,
               src_v, dst_v, cnt_v,
               sidx0, didx0, rows0, sidx1, didx1, rows1,
               shared, sem0, sem1):
    c = lax.axis_index("c")
    s = lax.axis_index("s")
    wid = c * 16 + s

    pltpu.sync_copy(hs_hbm.at[pl.ds(wid * 128, 128), :],
                    shared.at[pl.ds(s * 128, 128), :])
    pltpu.sync_copy(cnt_hbm.at[pl.ds(wid * 16, 16)], cnt_v)
    n = jnp.max(cnt_v[...], axis=0)
    pltpu.sync_copy(esrc_hbm.at[wid], src_v.at[pl.ds(0, CAPE)])
    pltpu.sync_copy(edst_hbm.at[wid], dst_v.at[pl.ds(0, CAPE)])

    _agg_loop(hs_hbm, src_v, dst_v, n, shared,
              sidx0, didx0, rows0, sem0, sidx1, didx1, rows1, sem1)
    pltpu.sync_copy(shared.at[pl.ds(s * 128, 128), :],
                    agg_hbm.at[pl.ds(wid * 128, 128), :])


def _aggregate2(hs, esrc, edst, cnt):
    mesh = plsc.VectorSubcoreMesh(core_axis_name="c", subcore_axis_name="s")
    return pl.kernel(
        _agg2_body,
        out_type=jax.ShapeDtypeStruct((N, D), jnp.float32),
        mesh=mesh,
        scratch_types=[
            pltpu.VMEM((CAPE + 16,), jnp.int32),
            pltpu.VMEM((CAPE + 16,), jnp.int32),
            pltpu.VMEM((16,), jnp.int32),
            pltpu.VMEM((CHUNK,), jnp.int32),
            pltpu.VMEM((CHUNK,), jnp.int32),
            pltpu.VMEM((CHUNK, D), jnp.float32),
            pltpu.VMEM((CHUNK,), jnp.int32),
            pltpu.VMEM((CHUNK,), jnp.int32),
            pltpu.VMEM((CHUNK, D), jnp.float32),
            pltpu.VMEM_SHARED((HALF + 16, D), jnp.float32),
            pltpu.SemaphoreType.DMA,
            pltpu.SemaphoreType.DMA,
        ],
        compiler_params=pltpu.CompilerParams(needs_layout_passes=False),
    )(hs, esrc, edst, cnt)


def _prep_body(f_ref, no_ref, out_ref):
    out_ref[...] = f_ref[...] * no_ref[...]


def _prep(features, no_col):
    grid = (4,)
    blk = N // 4
    return pl.pallas_call(
        _prep_body,
        grid=grid,
        in_specs=[
            pl.BlockSpec((blk, D), lambda i: (i, 0)),
            pl.BlockSpec((blk, 1), lambda i: (i, 0)),
        ],
        out_specs=pl.BlockSpec((blk, D), lambda i: (i, 0)),
        out_shape=jax.ShapeDtypeStruct((N, D), jnp.float32),
    )(features, no_col)


def _post_body(agg_ref, ni_ref, no_ref, w_ref, b_ref, h_ref, hs_ref):
    hd = agg_ref[...] * ni_ref[...]
    h = jax.nn.relu(
        jnp.dot(hd, w_ref[...], preferred_element_type=jnp.float32)
        + b_ref[...])
    h_ref[...] = h
    hs_ref[...] = h * no_ref[...]


def _post(agg, ni_col, no_col, W, b):
    grid = (4,)
    blk = N // 4
    return pl.pallas_call(
        _post_body,
        grid=grid,
        in_specs=[
            pl.BlockSpec((blk, D), lambda i: (i, 0)),
            pl.BlockSpec((blk, 1), lambda i: (i, 0)),
            pl.BlockSpec((blk, 1), lambda i: (i, 0)),
            pl.BlockSpec((D, D), lambda i: (0, 0)),
            pl.BlockSpec((1, D), lambda i: (0, 0)),
        ],
        out_specs=[
            pl.BlockSpec((blk, D), lambda i: (i, 0)),
            pl.BlockSpec((blk, D), lambda i: (i, 0)),
        ],
        out_shape=[
            jax.ShapeDtypeStruct((N, D), jnp.float32),
            jax.ShapeDtypeStruct((N, D), jnp.float32),
        ],
    )(agg, ni_col, no_col, W, b[None, :])


def kernel(A, features, threshold, W1, b1, W2, b2):
    thr = jnp.asarray(threshold, jnp.float32).reshape(1)
    packed, dout, din = _pack_and_degrees(A, thr)
    no_col = lax.rsqrt(dout[0] + 1.0)[:, None]
    ni_col = lax.rsqrt(din[0] + 1.0)[:, None]
    hs0 = _prep(features, no_col)
    esrc, edst, cnt, agg1 = _extract_agg(packed, hs0)
    h1, hs1 = _post(agg1, ni_col, no_col, W1, b1)
    agg2 = _aggregate2(hs1, esrc, edst, cnt)
    h2, _ = _post(agg2, ni_col, no_col, W2, b2)
    return (h1, h2)


# speedup vs baseline: 1.1234x; 1.0021x over previous
import jax
import jax.numpy as jnp
from jax import lax
from jax.experimental import pallas as pl
from jax.experimental.pallas import tpu as pltpu
from jax.experimental.pallas import tpu_sc as plsc

N = 4096
D = 128
NG32 = N // 32
NTILES = 32
GPT = NG32 // NTILES
CAPW = 4096
CAPE = 4096
CHUNK = 128
HALF = N // 2
DUMMY = HALF

BLK1 = 256


def _pack_body(thr_ref, a_ref, q_ref, pk_ref, dout_ref, din_ref):
    i = pl.program_id(0)
    mask = (a_ref[...] >= thr_ref[0]).astype(jnp.float32)
    dout_ref[...] = jnp.sum(mask, axis=1)[None, :]

    @pl.when(i == 0)
    def _():
        din_ref[...] = jnp.zeros_like(din_ref)

    din_ref[...] += jnp.sum(mask, axis=0)[None, :]
    pk = lax.dot_general(q_ref[...], mask.astype(jnp.bfloat16),
                         (((1,), (1,)), ((), ())),
                         preferred_element_type=jnp.float32).astype(jnp.int32)
    lo = lax.slice(pk, (0, 0), (NG32, BLK1))
    hi = lax.slice(pk, (NG32, 0), (2 * NG32, BLK1))
    pk_ref[...] = lo | lax.shift_left(hi, 16)


def _pack_and_degrees(A, thr):
    col = lax.iota(jnp.int32, N)
    gp = lax.iota(jnp.int32, 2 * NG32)[:, None]
    grp16 = jnp.where(gp < NG32, 2 * gp, 2 * (gp - NG32) + 1)
    q = jnp.where((col[None, :] // 16) == grp16,
                  jnp.exp2((col % 16).astype(jnp.float32))[None, :],
                  0.0).astype(jnp.bfloat16)
    grid = (N // BLK1,)
    return pl.pallas_call(
        _pack_body,
        grid=grid,
        in_specs=[
            pl.BlockSpec(memory_space=pltpu.SMEM),
            pl.BlockSpec((BLK1, N), lambda i: (i, 0)),
            pl.BlockSpec((2 * NG32, N), lambda i: (0, 0)),
        ],
        out_specs=[
            pl.BlockSpec((NG32, BLK1), lambda i: (0, i)),
            pl.BlockSpec((1, BLK1), lambda i: (0, i)),
            pl.BlockSpec((1, N), lambda i: (0, 0)),
        ],
        out_shape=[
            jax.ShapeDtypeStruct((NG32, N), jnp.int32),
            jax.ShapeDtypeStruct((1, N), jnp.float32),
            jax.ShapeDtypeStruct((1, N), jnp.float32),
        ],
        compiler_params=pltpu.CompilerParams(
            dimension_semantics=("arbitrary",),
        ),
    )(thr, A, q)


def _agg_loop(hs_hbm, src_v, dst_v, n, shared,
              sidx0, didx0, rows0, sem0, sidx1, didx1, rows1, sem1):
    nch = (n + CHUNK - 1) // CHUNK

    def load_idx(ci, sref, dref):
        for k in range(CHUNK // 16):
            sref[pl.ds(k * 16, 16)] = src_v[pl.ds(ci * CHUNK + k * 16, 16)]
            dref[pl.ds(k * 16, 16)] = dst_v[pl.ds(ci * CHUNK + k * 16, 16)]

    @pl.when(nch > 0)
    def _():
        load_idx(0, sidx0, didx0)
        pltpu.async_copy(hs_hbm.at[sidx0], rows0, sem0)

    def body(ci, _):
        p = lax.rem(ci, 2)

        @pl.when((p == 0) & (ci + 1 < nch))
        def _():
            load_idx(ci + 1, sidx1, didx1)
            pltpu.async_copy(hs_hbm.at[sidx1], rows1, sem1)

        @pl.when((p == 1) & (ci + 1 < nch))
        def _():
            load_idx(ci + 1, sidx0, didx0)
            pltpu.async_copy(hs_hbm.at[sidx0], rows0, sem0)

        @pl.when(p == 0)
        def _():
            pltpu.make_async_copy(hs_hbm.at[pl.ds(0, CHUNK), :], rows0,
                                  sem0).wait()
            pltpu.sync_copy(rows0, shared.at[didx0], add=True)

        @pl.when(p == 1)
        def _():
            pltpu.make_async_copy(hs_hbm.at[pl.ds(0, CHUNK), :], rows1,
                                  sem1).wait()
            pltpu.sync_copy(rows1, shared.at[didx1], add=True)

        return 0

    lax.fori_loop(0, nch, body, 0)


def _extract_agg_body(pk_hbm, hs_hbm, esrc_hbm, edst_hbm, cnt_hbm, agg_hbm,
                      pk_v, src_v, dst_v, nzw_v, nzi_v, out16_v,
                      sidx0, didx0, rows0, sidx1, didx1, rows1,
                      shared, sem0, sem1):
    c = lax.axis_index("c")
    s = lax.axis_index("s")
    wid = c * 16 + s
    g0 = wid * GPT
    iota = lax.iota(jnp.int32, 16)

    pltpu.sync_copy(pk_hbm.at[pl.ds(g0, GPT), :], pk_v)
    pltpu.sync_copy(hs_hbm.at[pl.ds(wid * 128, 128), :],
                    shared.at[pl.ds(s * 128, 128), :])

    scope_a = jax.named_scope("scan_words")
    scope_a.__enter__()
    offv = jnp.zeros((16,), jnp.int32)
    for gl in range(GPT):
        def body_a(i, offv, gl=gl):
            v = pk_v[gl, pl.ds(i * 16, 16)]
            mm = v != 0
            mi = mm.astype(jnp.int32)
            excl = plsc.cumsum(mi) - mi
            dest = jnp.where(mm, offv + excl, CAPW)
            plsc.store_scatter(nzw_v, [dest], v)
            plsc.store_scatter(nzi_v, [dest], gl * N + i * 16 + iota)
            cnt = plsc.all_reduce_population_count(mm)
            return jnp.minimum(offv + cnt, CAPW - 16)
        offv = lax.fori_loop(0, N // 16, body_a, offv)

    scope_a.__exit__(None, None, None)
    scope_b = jax.named_scope("peel_bits")
    scope_b.__enter__()
    nw = jnp.max(offv, axis=0)
    dbase0 = g0 * 32 - c * HALF

    def body_b(j, ecv):
        lanem = iota < (nw - j * 16)
        w = jnp.where(lanem, nzw_v[pl.ds(j * 16, 16)], 0)
        fi = nzi_v[pl.ds(j * 16, 16)]
        r = fi & (N - 1)
        gl = lax.shift_right_logical(fi, 12)
        dstbase = dbase0 + gl * 32

        def peel_cond(carry):
            return jnp.any(carry[0] != 0)

        def peel(carry):
            w, ecv = carry
            b = w & (-w)
            mm = b != 0
            bf = b.astype(jnp.float32)
            bi = (lax.shift_right_logical(
                lax.bitcast_convert_type(bf, jnp.int32), 23) & 0xFF) - 127
            mi = mm.astype(jnp.int32)
            excl = plsc.cumsum(mi) - mi
            dest = jnp.where(mm, ecv + excl, CAPE)
            plsc.store_scatter(src_v, [dest], r)
            plsc.store_scatter(dst_v, [dest], dstbase + bi)
            cnt = plsc.all_reduce_population_count(mm)
            ecv = jnp.minimum(ecv + cnt, CAPE - 16)
            return (w & (w - 1), ecv)

        _, ecv = lax.while_loop(peel_cond, peel, (w, ecv))
        return ecv

    ecv = lax.fori_loop(0, (nw + 15) // 16, body_b,
                        jnp.zeros((16,), jnp.int32))
    ec = jnp.max(ecv, axis=0)

    target = ((ec + CHUNK - 1) // CHUNK) * CHUNK

    def pad(o):
        src_v[pl.ds(o, 16)] = jnp.zeros((16,), jnp.int32)
        dst_v[pl.ds(o, 16)] = jnp.full((16,), DUMMY, jnp.int32)
        return o + 16

    lax.while_loop(lambda o: o < target, pad, ec)

    out16_v[...] = jnp.full((16,), ec, jnp.int32)
    pltpu.sync_copy(out16_v, cnt_hbm.at[pl.ds(wid * 16, 16)])
    pltpu.sync_copy(src_v.at[pl.ds(0, CAPE)], esrc_hbm.at[wid])
    pltpu.sync_copy(dst_v.at[pl.ds(0, CAPE)], edst_hbm.at[wid])

    scope_b.__exit__(None, None, None)
    with jax.named_scope("agg1"):
        _agg_loop(hs_hbm, src_v, dst_v, ec, shared,
                  sidx0, didx0, rows0, sem0, sidx1, didx1, rows1, sem1)
    pltpu.sync_copy(shared.at[pl.ds(s * 128, 128), :],
                    agg_hbm.at[pl.ds(wid * 128, 128), :])


def _extract_agg(packed, hs0):
    mesh = plsc.VectorSubcoreMesh(core_axis_name="c", subcore_axis_name="s")
    return pl.kernel(
        _extract_agg_body,
        out_type=[
            jax.ShapeDtypeStruct((NTILES, CAPE), jnp.int32),
            jax.ShapeDtypeStruct((NTILES, CAPE), jnp.int32),
            jax.ShapeDtypeStruct((NTILES * 16,), jnp.int32),
            jax.ShapeDtypeStruct((N, D), jnp.float32),
        ],
        mesh=mesh,
        scratch_types=[
            pltpu.VMEM((GPT, N), jnp.int32),
            pltpu.VMEM((CAPE + 16,), jnp.int32),
            pltpu.VMEM((CAPE + 16,), jnp.int32),
            pltpu.VMEM((CAPW + 16,), jnp.int32),
            pltpu.VMEM((CAPW + 16,), jnp.int32),
            pltpu.VMEM((16,), jnp.int32),
            pltpu.VMEM((CHUNK,), jnp.int32),
            pltpu.VMEM((CHUNK,), jnp.int32),
            pltpu.VMEM((CHUNK, D), jnp.float32),
            pltpu.VMEM((CHUNK,), jnp.int32),
            pltpu.VMEM((CHUNK,), jnp.int32),
            pltpu.VMEM((CHUNK, D), jnp.float32),
            pltpu.VMEM_SHARED((HALF + 16, D), jnp.float32),
            pltpu.SemaphoreType.DMA,
            pltpu.SemaphoreType.DMA,
        ],
        compiler_params=pltpu.CompilerParams(needs_layout_passes=False),
    )(packed, hs0)


def _agg2_body(hs_hbm, esrc_hbm, edst_hbm, cnt_hbm, agg_hbm,
               src_v, dst_v, cnt_v,
               sidx0, didx0, rows0, sidx1, didx1, rows1,
               shared, sem0, sem1):
    c = lax.axis_index("c")
    s = lax.axis_index("s")
    wid = c * 16 + s

    pltpu.sync_copy(hs_hbm.at[pl.ds(wid * 128, 128), :],
                    shared.at[pl.ds(s * 128, 128), :])
    pltpu.sync_copy(cnt_hbm.at[pl.ds(wid * 16, 16)], cnt_v)
    n = jnp.max(cnt_v[...], axis=0)
    pltpu.sync_copy(esrc_hbm.at[wid], src_v.at[pl.ds(0, CAPE)])
    pltpu.sync_copy(edst_hbm.at[wid], dst_v.at[pl.ds(0, CAPE)])

    _agg_loop(hs_hbm, src_v, dst_v, n, shared,
              sidx0, didx0, rows0, sem0, sidx1, didx1, rows1, sem1)
    pltpu.sync_copy(shared.at[pl.ds(s * 128, 128), :],
                    agg_hbm.at[pl.ds(wid * 128, 128), :])


def _aggregate2(hs, esrc, edst, cnt):
    mesh = plsc.VectorSubcoreMesh(core_axis_name="c", subcore_axis_name="s")
    return pl.kernel(
        _agg2_body,
        out_type=jax.ShapeDtypeStruct((N, D), jnp.float32),
        mesh=mesh,
        scratch_types=[
            pltpu.VMEM((CAPE + 16,), jnp.int32),
            pltpu.VMEM((CAPE + 16,), jnp.int32),
            pltpu.VMEM((16,), jnp.int32),
            pltpu.VMEM((CHUNK,), jnp.int32),
            pltpu.VMEM((CHUNK,), jnp.int32),
            pltpu.VMEM((CHUNK, D), jnp.float32),
            pltpu.VMEM((CHUNK,), jnp.int32),
            pltpu.VMEM((CHUNK,), jnp.int32),
            pltpu.VMEM((CHUNK, D), jnp.float32),
            pltpu.VMEM_SHARED((HALF + 16, D), jnp.float32),
            pltpu.SemaphoreType.DMA,
            pltpu.SemaphoreType.DMA,
        ],
        compiler_params=pltpu.CompilerParams(needs_layout_passes=False),
    )(hs, esrc, edst, cnt)


def _prep_body(f_ref, no_ref, out_ref):
    out_ref[...] = f_ref[...] * no_ref[...]


def _prep(features, no_col):
    grid = (4,)
    blk = N // 4
    return pl.pallas_call(
        _prep_body,
        grid=grid,
        in_specs=[
            pl.BlockSpec((blk, D), lambda i: (i, 0)),
            pl.BlockSpec((blk, 1), lambda i: (i, 0)),
        ],
        out_specs=pl.BlockSpec((blk, D), lambda i: (i, 0)),
        out_shape=jax.ShapeDtypeStruct((N, D), jnp.float32),
    )(features, no_col)


def _post_body(agg_ref, ni_ref, no_ref, w_ref, b_ref, h_ref, hs_ref):
    hd = agg_ref[...] * ni_ref[...]
    h = jax.nn.relu(
        jnp.dot(hd, w_ref[...], preferred_element_type=jnp.float32)
        + b_ref[...])
    h_ref[...] = h
    hs_ref[...] = h * no_ref[...]


def _post(agg, ni_col, no_col, W, b):
    grid = (4,)
    blk = N // 4
    return pl.pallas_call(
        _post_body,
        grid=grid,
        in_specs=[
            pl.BlockSpec((blk, D), lambda i: (i, 0)),
            pl.BlockSpec((blk, 1), lambda i: (i, 0)),
            pl.BlockSpec((blk, 1), lambda i: (i, 0)),
            pl.BlockSpec((D, D), lambda i: (0, 0)),
            pl.BlockSpec((1, D), lambda i: (0, 0)),
        ],
        out_specs=[
            pl.BlockSpec((blk, D), lambda i: (i, 0)),
            pl.BlockSpec((blk, D), lambda i: (i, 0)),
        ],
        out_shape=[
            jax.ShapeDtypeStruct((N, D), jnp.float32),
            jax.ShapeDtypeStruct((N, D), jnp.float32),
        ],
    )(agg, ni_col, no_col, W, b[None, :])


def kernel(A, features, threshold, W1, b1, W2, b2):
    thr = jnp.asarray(threshold, jnp.float32).reshape(1)
    packed, dout, din = _pack_and_degrees(A, thr)
    no_col = lax.rsqrt(dout[0] + 1.0)[:, None]
    ni_col = lax.rsqrt(din[0] + 1.0)[:, None]
    hs0 = _prep(features, no_col)
    esrc, edst, cnt, agg1 = _extract_agg(packed, hs0)
    h1, hs1 = _post(agg1, ni_col, no_col, W1, b1)
    agg2 = _aggregate2(hs1, esrc, edst, cnt)
    h2, _ = _post(agg2, ni_col, no_col, W2, b2)
    return (h1, h2)


# speedup vs baseline: 2.3622x; 2.1027x over previous
import jax
import jax.numpy as jnp
from jax import lax
from jax.experimental import pallas as pl
from jax.experimental.pallas import tpu as pltpu
from jax.experimental.pallas import tpu_sc as plsc

N = 4096
D = 128
NG32 = N // 32
NTILES = 32
GPT = NG32 // NTILES
CAPW = 4096
CAPE = 4096
CHUNK = 128
HALF = N // 2
DUMMY = HALF

BLK1 = 256


def _pack_body(thr_ref, a_ref, q_ref, pk_ref, dout_ref, din_ref):
    i = pl.program_id(0)
    mask = (a_ref[...] >= thr_ref[0]).astype(jnp.float32)
    dout_ref[...] = jnp.sum(mask, axis=1)[None, :]

    @pl.when(i == 0)
    def _():
        din_ref[...] = jnp.zeros_like(din_ref)

    din_ref[...] += jnp.sum(mask, axis=0)[None, :]
    pk = lax.dot_general(q_ref[...], mask.astype(jnp.bfloat16),
                         (((1,), (1,)), ((), ())),
                         preferred_element_type=jnp.float32).astype(jnp.int32)
    lo = lax.slice(pk, (0, 0), (NG32, BLK1))
    hi = lax.slice(pk, (NG32, 0), (2 * NG32, BLK1))
    pk_ref[...] = lo | lax.shift_left(hi, 16)


def _pack_and_degrees(A, thr):
    col = lax.iota(jnp.int32, N)
    gp = lax.iota(jnp.int32, 2 * NG32)[:, None]
    grp16 = jnp.where(gp < NG32, 2 * gp, 2 * (gp - NG32) + 1)
    q = jnp.where((col[None, :] // 16) == grp16,
                  jnp.exp2((col % 16).astype(jnp.float32))[None, :],
                  0.0).astype(jnp.bfloat16)
    grid = (N // BLK1,)
    return pl.pallas_call(
        _pack_body,
        grid=grid,
        in_specs=[
            pl.BlockSpec(memory_space=pltpu.SMEM),
            pl.BlockSpec((BLK1, N), lambda i: (i, 0)),
            pl.BlockSpec((2 * NG32, N), lambda i: (0, 0)),
        ],
        out_specs=[
            pl.BlockSpec((NG32, BLK1), lambda i: (0, i)),
            pl.BlockSpec((1, BLK1), lambda i: (0, i)),
            pl.BlockSpec((1, N), lambda i: (0, 0)),
        ],
        out_shape=[
            jax.ShapeDtypeStruct((NG32, N), jnp.int32),
            jax.ShapeDtypeStruct((1, N), jnp.float32),
            jax.ShapeDtypeStruct((1, N), jnp.float32),
        ],
        compiler_params=pltpu.CompilerParams(
            dimension_semantics=("arbitrary",),
        ),
    )(thr, A, q)


def _agg_loop(sh_hs, hs_hbm, src_v, dst_v, n, shared,
              sidx0, didx0, rows0, sem0, sidx1, didx1, rows1, sem1):
    nch = (n + CHUNK - 1) // CHUNK

    def load_idx(ci, sref, dref):
        for k in range(CHUNK // 16):
            sref[pl.ds(k * 16, 16)] = src_v[pl.ds(ci * CHUNK + k * 16, 16)]
            dref[pl.ds(k * 16, 16)] = dst_v[pl.ds(ci * CHUNK + k * 16, 16)]

    def drain(rows, sem):
        pltpu.make_async_copy(hs_hbm.at[pl.ds(0, CHUNK), :], rows, sem).wait()

    @pl.when(nch > 0)
    def _():
        load_idx(0, sidx0, didx0)
        pltpu.async_copy(sh_hs.at[sidx0], rows0, sem0)

    def body(ci, _):
        p = lax.rem(ci, 2)

        @pl.when((p == 0) & (ci + 1 < nch))
        def _():
            @pl.when(ci >= 1)
            def _():
                drain(rows1, sem1)
            load_idx(ci + 1, sidx1, didx1)
            pltpu.async_copy(sh_hs.at[sidx1], rows1, sem1)

        @pl.when((p == 1) & (ci + 1 < nch))
        def _():
            drain(rows0, sem0)
            load_idx(ci + 1, sidx0, didx0)
            pltpu.async_copy(sh_hs.at[sidx0], rows0, sem0)

        @pl.when(p == 0)
        def _():
            drain(rows0, sem0)
            pltpu.async_copy(rows0, shared.at[didx0], sem0, add=True)

        @pl.when(p == 1)
        def _():
            drain(rows1, sem1)
            pltpu.async_copy(rows1, shared.at[didx1], sem1, add=True)

        return 0

    lax.fori_loop(0, nch, body, 0)
    @pl.when(nch > 0)
    def _():
        p = lax.rem(nch - 1, 2)

        @pl.when(p == 0)
        def _():
            drain(rows0, sem0)

        @pl.when(p == 1)
        def _():
            drain(rows1, sem1)

    @pl.when(nch > 1)
    def _():
        p = lax.rem(nch - 2, 2)

        @pl.when(p == 0)
        def _():
            drain(rows0, sem0)

        @pl.when(p == 1)
        def _():
            drain(rows1, sem1)


def _extract_agg_body(pk_hbm, hs_hbm, esrc_hbm, edst_hbm, cnt_hbm, agg_hbm,
                      pk_v, src_v, dst_v, nzw_v, nzi_v, out16_v,
                      sidx0, didx0, rows0, sidx1, didx1, rows1,
                      shared, sh_hs, sem0, sem1, sem_st):
    c = lax.axis_index("c")
    s = lax.axis_index("s")
    wid = c * 16 + s
    g0 = wid * GPT
    iota = lax.iota(jnp.int32, 16)

    pltpu.sync_copy(pk_hbm.at[pl.ds(g0, GPT), :], pk_v)
    pltpu.async_copy(hs_hbm.at[pl.ds(s * 256, 256), :],
                     sh_hs.at[pl.ds(s * 256, 256), :], sem_st)
    pltpu.sync_copy(hs_hbm.at[pl.ds(wid * 128, 128), :],
                    shared.at[pl.ds(s * 128, 128), :])

    scope_a = jax.named_scope("scan_words")
    scope_a.__enter__()
    offv = jnp.zeros((16,), jnp.int32)
    for gl in range(GPT):
        def body_a(i, offv, gl=gl):
            v = pk_v[gl, pl.ds(i * 16, 16)]
            mm = v != 0
            mi = mm.astype(jnp.int32)
            excl = plsc.cumsum(mi) - mi
            dest = jnp.where(mm, offv + excl, CAPW)
            plsc.store_scatter(nzw_v, [dest], v)
            plsc.store_scatter(nzi_v, [dest], gl * N + i * 16 + iota)
            cnt = plsc.all_reduce_population_count(mm)
            return jnp.minimum(offv + cnt, CAPW - 16)
        offv = lax.fori_loop(0, N // 16, body_a, offv)

    scope_a.__exit__(None, None, None)
    scope_b = jax.named_scope("peel_bits")
    scope_b.__enter__()
    nw = jnp.max(offv, axis=0)
    dbase0 = g0 * 32 - c * HALF

    def body_b(j, ecv):
        lanem = iota < (nw - j * 16)
        w = jnp.where(lanem, nzw_v[pl.ds(j * 16, 16)], 0)
        fi = nzi_v[pl.ds(j * 16, 16)]
        r = fi & (N - 1)
        gl = lax.shift_right_logical(fi, 12)
        dstbase = dbase0 + gl * 32

        def peel_cond(carry):
            return jnp.any(carry[0] != 0)

        def peel(carry):
            w, ecv = carry
            b = w & (-w)
            mm = b != 0
            bf = b.astype(jnp.float32)
            bi = (lax.shift_right_logical(
                lax.bitcast_convert_type(bf, jnp.int32), 23) & 0xFF) - 127
            mi = mm.astype(jnp.int32)
            excl = plsc.cumsum(mi) - mi
            dest = jnp.where(mm, ecv + excl, CAPE)
            plsc.store_scatter(src_v, [dest], r)
            plsc.store_scatter(dst_v, [dest], dstbase + bi)
            cnt = plsc.all_reduce_population_count(mm)
            ecv = jnp.minimum(ecv + cnt, CAPE - 16)
            return (w & (w - 1), ecv)

        _, ecv = lax.while_loop(peel_cond, peel, (w, ecv))
        return ecv

    ecv = lax.fori_loop(0, (nw + 15) // 16, body_b,
                        jnp.zeros((16,), jnp.int32))
    ec = jnp.max(ecv, axis=0)

    target = ((ec + CHUNK - 1) // CHUNK) * CHUNK

    def pad(o):
        src_v[pl.ds(o, 16)] = jnp.zeros((16,), jnp.int32)
        dst_v[pl.ds(o, 16)] = jnp.full((16,), DUMMY, jnp.int32)
        return o + 16

    lax.while_loop(lambda o: o < target, pad, ec)

    out16_v[...] = jnp.full((16,), ec, jnp.int32)
    pltpu.sync_copy(out16_v, cnt_hbm.at[pl.ds(wid * 16, 16)])
    pltpu.sync_copy(src_v.at[pl.ds(0, CAPE)], esrc_hbm.at[wid])
    pltpu.sync_copy(dst_v.at[pl.ds(0, CAPE)], edst_hbm.at[wid])

    scope_b.__exit__(None, None, None)
    pltpu.make_async_copy(hs_hbm.at[pl.ds(0, 256), :],
                          sh_hs.at[pl.ds(0, 256), :], sem_st).wait()
    plsc.subcore_barrier()
    with jax.named_scope("agg1"):
        _agg_loop(sh_hs, hs_hbm, src_v, dst_v, ec, shared,
                  sidx0, didx0, rows0, sem0, sidx1, didx1, rows1, sem1)
    pltpu.sync_copy(shared.at[pl.ds(s * 128, 128), :],
                    agg_hbm.at[pl.ds(wid * 128, 128), :])


def _extract_agg(packed, hs0):
    mesh = plsc.VectorSubcoreMesh(core_axis_name="c", subcore_axis_name="s")
    return pl.kernel(
        _extract_agg_body,
        out_type=[
            jax.ShapeDtypeStruct((NTILES, CAPE), jnp.int32),
            jax.ShapeDtypeStruct((NTILES, CAPE), jnp.int32),
            jax.ShapeDtypeStruct((NTILES * 16,), jnp.int32),
            jax.ShapeDtypeStruct((N, D), jnp.float32),
        ],
        mesh=mesh,
        scratch_types=[
            pltpu.VMEM((GPT, N), jnp.int32),
            pltpu.VMEM((CAPE + 16,), jnp.int32),
            pltpu.VMEM((CAPE + 16,), jnp.int32),
            pltpu.VMEM((CAPW + 16,), jnp.int32),
            pltpu.VMEM((CAPW + 16,), jnp.int32),
            pltpu.VMEM((16,), jnp.int32),
            pltpu.VMEM((CHUNK,), jnp.int32),
            pltpu.VMEM((CHUNK,), jnp.int32),
            pltpu.VMEM((CHUNK, D), jnp.float32),
            pltpu.VMEM((CHUNK,), jnp.int32),
            pltpu.VMEM((CHUNK,), jnp.int32),
            pltpu.VMEM((CHUNK, D), jnp.float32),
            pltpu.VMEM_SHARED((HALF + 16, D), jnp.float32),
            pltpu.VMEM_SHARED((N, D), jnp.float32),
            pltpu.SemaphoreType.DMA,
            pltpu.SemaphoreType.DMA,
            pltpu.SemaphoreType.DMA,
        ],
        compiler_params=pltpu.CompilerParams(needs_layout_passes=False),
    )(packed, hs0)


def _agg2_body(hs_hbm, esrc_hbm, edst_hbm, cnt_hbm, agg_hbm,
               src_v, dst_v, cnt_v,
               sidx0, didx0, rows0, sidx1, didx1, rows1,
               shared, sh_hs, sem0, sem1, sem_st):
    c = lax.axis_index("c")
    s = lax.axis_index("s")
    wid = c * 16 + s

    pltpu.async_copy(hs_hbm.at[pl.ds(s * 256, 256), :],
                     sh_hs.at[pl.ds(s * 256, 256), :], sem_st)
    pltpu.sync_copy(hs_hbm.at[pl.ds(wid * 128, 128), :],
                    shared.at[pl.ds(s * 128, 128), :])
    pltpu.sync_copy(cnt_hbm.at[pl.ds(wid * 16, 16)], cnt_v)
    n = jnp.max(cnt_v[...], axis=0)
    pltpu.sync_copy(esrc_hbm.at[wid], src_v.at[pl.ds(0, CAPE)])
    pltpu.sync_copy(edst_hbm.at[wid], dst_v.at[pl.ds(0, CAPE)])
    pltpu.make_async_copy(hs_hbm.at[pl.ds(0, 256), :],
                          sh_hs.at[pl.ds(0, 256), :], sem_st).wait()
    plsc.subcore_barrier()

    _agg_loop(sh_hs, hs_hbm, src_v, dst_v, n, shared,
              sidx0, didx0, rows0, sem0, sidx1, didx1, rows1, sem1)
    pltpu.sync_copy(shared.at[pl.ds(s * 128, 128), :],
                    agg_hbm.at[pl.ds(wid * 128, 128), :])


def _aggregate2(hs, esrc, edst, cnt):
    mesh = plsc.VectorSubcoreMesh(core_axis_name="c", subcore_axis_name="s")
    return pl.kernel(
        _agg2_body,
        out_type=jax.ShapeDtypeStruct((N, D), jnp.float32),
        mesh=mesh,
        scratch_types=[
            pltpu.VMEM((CAPE + 16,), jnp.int32),
            pltpu.VMEM((CAPE + 16,), jnp.int32),
            pltpu.VMEM((16,), jnp.int32),
            pltpu.VMEM((CHUNK,), jnp.int32),
            pltpu.VMEM((CHUNK,), jnp.int32),
            pltpu.VMEM((CHUNK, D), jnp.float32),
            pltpu.VMEM((CHUNK,), jnp.int32),
            pltpu.VMEM((CHUNK,), jnp.int32),
            pltpu.VMEM((CHUNK, D), jnp.float32),
            pltpu.VMEM_SHARED((HALF + 16, D), jnp.float32),
            pltpu.VMEM_SHARED((N, D), jnp.float32),
            pltpu.SemaphoreType.DMA,
            pltpu.SemaphoreType.DMA,
            pltpu.SemaphoreType.DMA,
        ],
        compiler_params=pltpu.CompilerParams(needs_layout_passes=False),
    )(hs, esrc, edst, cnt)


def _prep_body(f_ref, no_ref, out_ref):
    out_ref[...] = f_ref[...] * no_ref[...]


def _prep(features, no_col):
    grid = (4,)
    blk = N // 4
    return pl.pallas_call(
        _prep_body,
        grid=grid,
        in_specs=[
            pl.BlockSpec((blk, D), lambda i: (i, 0)),
            pl.BlockSpec((blk, 1), lambda i: (i, 0)),
        ],
        out_specs=pl.BlockSpec((blk, D), lambda i: (i, 0)),
        out_shape=jax.ShapeDtypeStruct((N, D), jnp.float32),
    )(features, no_col)


def _post_body(agg_ref, ni_ref, no_ref, w_ref, b_ref, h_ref, hs_ref):
    hd = agg_ref[...] * ni_ref[...]
    h = jax.nn.relu(
        jnp.dot(hd, w_ref[...], preferred_element_type=jnp.float32)
        + b_ref[...])
    h_ref[...] = h
    hs_ref[...] = h * no_ref[...]


def _post(agg, ni_col, no_col, W, b):
    grid = (4,)
    blk = N // 4
    return pl.pallas_call(
        _post_body,
        grid=grid,
        in_specs=[
            pl.BlockSpec((blk, D), lambda i: (i, 0)),
            pl.BlockSpec((blk, 1), lambda i: (i, 0)),
            pl.BlockSpec((blk, 1), lambda i: (i, 0)),
            pl.BlockSpec((D, D), lambda i: (0, 0)),
            pl.BlockSpec((1, D), lambda i: (0, 0)),
        ],
        out_specs=[
            pl.BlockSpec((blk, D), lambda i: (i, 0)),
            pl.BlockSpec((blk, D), lambda i: (i, 0)),
        ],
        out_shape=[
            jax.ShapeDtypeStruct((N, D), jnp.float32),
            jax.ShapeDtypeStruct((N, D), jnp.float32),
        ],
    )(agg, ni_col, no_col, W, b[None, :])


def kernel(A, features, threshold, W1, b1, W2, b2):
    thr = jnp.asarray(threshold, jnp.float32).reshape(1)
    packed, dout, din = _pack_and_degrees(A, thr)
    no_col = lax.rsqrt(dout[0] + 1.0)[:, None]
    ni_col = lax.rsqrt(din[0] + 1.0)[:, None]
    hs0 = _prep(features, no_col)
    esrc, edst, cnt, agg1 = _extract_agg(packed, hs0)
    h1, hs1 = _post(agg1, ni_col, no_col, W1, b1)
    agg2 = _aggregate2(hs1, esrc, edst, cnt)
    h2, _ = _post(agg2, ni_col, no_col, W2, b2)
    return (h1, h2)


# speedup vs baseline: 2.4093x; 1.0199x over previous
import jax
import jax.numpy as jnp
from jax import lax
from jax.experimental import pallas as pl
from jax.experimental.pallas import tpu as pltpu
from jax.experimental.pallas import tpu_sc as plsc

N = 4096
D = 128
NG32 = N // 32
NTILES = 32
GPT = NG32 // NTILES
CAPW = 4096
CAPE = 4096
CHUNK = 128
HALF = N // 2
DUMMY = HALF

BLK1 = 256


def _pack_body(thr_ref, a_ref, qt_ref, f_ref, pk_ref, dout_ref, din_ref,
               hs_ref):
    i = pl.program_id(0)
    mask = (a_ref[...] >= thr_ref[0]).astype(jnp.float32).astype(jnp.bfloat16)
    pkt = lax.dot_general(mask, qt_ref[...], (((1,), (0,)), ((), ())),
                          preferred_element_type=jnp.float32)
    dout = jnp.sum(lax.slice(pkt, (0, 2 * NG32), (BLK1, 2 * NG32 + 8)),
                   axis=1, keepdims=True)
    dout_ref[...] = dout
    hs_ref[...] = f_ref[...] * lax.rsqrt(dout + 1.0)

    @pl.when(i == 0)
    def _():
        din_ref[...] = jnp.zeros_like(din_ref)

    ones8 = jnp.full((8, BLK1), jnp.bfloat16(1), jnp.bfloat16)
    din_ref[...] += lax.dot_general(ones8, mask, (((1,), (0,)), ((), ())),
                                    preferred_element_type=jnp.float32)

    pk32 = jnp.transpose(
        lax.slice(pkt, (0, 0), (BLK1, 2 * NG32)).astype(jnp.int32), (1, 0))
    lo = lax.slice(pk32, (0, 0), (NG32, BLK1))
    hi = lax.slice(pk32, (NG32, 0), (2 * NG32, BLK1))
    pk_ref[...] = lo | lax.shift_left(hi, 16)


def _pack_and_degrees(A, features, thr):
    col = lax.iota(jnp.int32, N)
    gp = lax.iota(jnp.int32, 2 * NG32)[:, None]
    grp16 = jnp.where(gp < NG32, 2 * gp, 2 * (gp - NG32) + 1)
    q = jnp.where((col[None, :] // 16) == grp16,
                  jnp.exp2((col % 16).astype(jnp.float32))[None, :], 0.0)
    q = jnp.concatenate(
        [q, jnp.ones((1, N), jnp.float32), jnp.zeros((7, N), jnp.float32)],
        axis=0)
    qt = q.T.astype(jnp.bfloat16)
    grid = (N // BLK1,)
    return pl.pallas_call(
        _pack_body,
        grid=grid,
        in_specs=[
            pl.BlockSpec(memory_space=pltpu.SMEM),
            pl.BlockSpec((BLK1, N), lambda i: (i, 0)),
            pl.BlockSpec((N, 2 * NG32 + 8), lambda i: (0, 0)),
            pl.BlockSpec((BLK1, D), lambda i: (i, 0)),
        ],
        out_specs=[
            pl.BlockSpec((NG32, BLK1), lambda i: (0, i)),
            pl.BlockSpec((BLK1, 1), lambda i: (i, 0)),
            pl.BlockSpec((8, N), lambda i: (0, 0)),
            pl.BlockSpec((BLK1, D), lambda i: (i, 0)),
        ],
        out_shape=[
            jax.ShapeDtypeStruct((NG32, N), jnp.int32),
            jax.ShapeDtypeStruct((N, 1), jnp.float32),
            jax.ShapeDtypeStruct((8, N), jnp.float32),
            jax.ShapeDtypeStruct((N, D), jnp.float32),
        ],
        compiler_params=pltpu.CompilerParams(
            dimension_semantics=("arbitrary",),
        ),
    )(thr, A, qt, features)


def _agg_loop(sh_hs, hs_hbm, src_v, dst_v, n, shared,
              sidx0, didx0, rows0, sem0, sidx1, didx1, rows1, sem1):
    nch = (n + CHUNK - 1) // CHUNK

    def load_idx(ci, sref, dref):
        for k in range(CHUNK // 16):
            sref[pl.ds(k * 16, 16)] = src_v[pl.ds(ci * CHUNK + k * 16, 16)]
            dref[pl.ds(k * 16, 16)] = dst_v[pl.ds(ci * CHUNK + k * 16, 16)]

    def drain(rows, sem):
        pltpu.make_async_copy(hs_hbm.at[pl.ds(0, CHUNK), :], rows, sem).wait()

    @pl.when(nch > 0)
    def _():
        load_idx(0, sidx0, didx0)
        pltpu.async_copy(sh_hs.at[sidx0], rows0, sem0)

    def body(ci, _):
        p = lax.rem(ci, 2)

        @pl.when((p == 0) & (ci + 1 < nch))
        def _():
            @pl.when(ci >= 1)
            def _():
                drain(rows1, sem1)
            load_idx(ci + 1, sidx1, didx1)
            pltpu.async_copy(sh_hs.at[sidx1], rows1, sem1)

        @pl.when((p == 1) & (ci + 1 < nch))
        def _():
            drain(rows0, sem0)
            load_idx(ci + 1, sidx0, didx0)
            pltpu.async_copy(sh_hs.at[sidx0], rows0, sem0)

        @pl.when(p == 0)
        def _():
            drain(rows0, sem0)
            pltpu.async_copy(rows0, shared.at[didx0], sem0, add=True)

        @pl.when(p == 1)
        def _():
            drain(rows1, sem1)
            pltpu.async_copy(rows1, shared.at[didx1], sem1, add=True)

        return 0

    lax.fori_loop(0, nch, body, 0)
    @pl.when(nch > 0)
    def _():
        p = lax.rem(nch - 1, 2)

        @pl.when(p == 0)
        def _():
            drain(rows0, sem0)

        @pl.when(p == 1)
        def _():
            drain(rows1, sem1)

    @pl.when(nch > 1)
    def _():
        p = lax.rem(nch - 2, 2)

        @pl.when(p == 0)
        def _():
            drain(rows0, sem0)

        @pl.when(p == 1)
        def _():
            drain(rows1, sem1)


def _extract_agg_body(pk_hbm, hs_hbm, esrc_hbm, edst_hbm, cnt_hbm, agg_hbm,
                      pk_v, src_v, dst_v, nzw_v, nzi_v, out16_v,
                      sidx0, didx0, rows0, sidx1, didx1, rows1,
                      shared, sh_hs, sem0, sem1, sem_st):
    c = lax.axis_index("c")
    s = lax.axis_index("s")
    wid = c * 16 + s
    g0 = wid * GPT
    iota = lax.iota(jnp.int32, 16)

    pltpu.sync_copy(pk_hbm.at[pl.ds(g0, GPT), :], pk_v)
    pltpu.async_copy(hs_hbm.at[pl.ds(s * 256, 256), :],
                     sh_hs.at[pl.ds(s * 256, 256), :], sem_st)
    pltpu.sync_copy(hs_hbm.at[pl.ds(wid * 128, 128), :],
                    shared.at[pl.ds(s * 128, 128), :])

    offv = jnp.zeros((16,), jnp.int32)
    for gl in range(GPT):
        def body_a(i, offv, gl=gl):
            v = pk_v[gl, pl.ds(i * 16, 16)]
            mm = v != 0
            mi = mm.astype(jnp.int32)
            excl = plsc.cumsum(mi) - mi
            dest = jnp.where(mm, offv + excl, CAPW)
            plsc.store_scatter(nzw_v, [dest], v)
            plsc.store_scatter(nzi_v, [dest], gl * N + i * 16 + iota)
            cnt = plsc.all_reduce_population_count(mm)
            return jnp.minimum(offv + cnt, CAPW - 16)
        offv = lax.fori_loop(0, N // 16, body_a, offv)

    nw = jnp.max(offv, axis=0)
    dbase0 = g0 * 32 - c * HALF

    def body_b(j, ecv):
        lanem = iota < (nw - j * 16)
        w = jnp.where(lanem, nzw_v[pl.ds(j * 16, 16)], 0)
        fi = nzi_v[pl.ds(j * 16, 16)]
        r = fi & (N - 1)
        gl = lax.shift_right_logical(fi, 12)
        dstbase = dbase0 + gl * 32

        def peel_cond(carry):
            return jnp.any(carry[0] != 0)

        def peel(carry):
            w, ecv = carry
            b = w & (-w)
            mm = b != 0
            bf = b.astype(jnp.float32)
            bi = (lax.shift_right_logical(
                lax.bitcast_convert_type(bf, jnp.int32), 23) & 0xFF) - 127
            mi = mm.astype(jnp.int32)
            excl = plsc.cumsum(mi) - mi
            dest = jnp.where(mm, ecv + excl, CAPE)
            plsc.store_scatter(src_v, [dest], r)
            plsc.store_scatter(dst_v, [dest], dstbase + bi)
            cnt = plsc.all_reduce_population_count(mm)
            ecv = jnp.minimum(ecv + cnt, CAPE - 16)
            return (w & (w - 1), ecv)

        _, ecv = lax.while_loop(peel_cond, peel, (w, ecv))
        return ecv

    ecv = lax.fori_loop(0, (nw + 15) // 16, body_b,
                        jnp.zeros((16,), jnp.int32))
    ec = jnp.max(ecv, axis=0)

    target = ((ec + CHUNK - 1) // CHUNK) * CHUNK

    def pad(o):
        src_v[pl.ds(o, 16)] = jnp.zeros((16,), jnp.int32)
        dst_v[pl.ds(o, 16)] = jnp.full((16,), DUMMY, jnp.int32)
        return o + 16

    lax.while_loop(lambda o: o < target, pad, ec)

    out16_v[...] = jnp.full((16,), ec, jnp.int32)
    pltpu.sync_copy(out16_v, cnt_hbm.at[pl.ds(wid * 16, 16)])
    pltpu.sync_copy(src_v.at[pl.ds(0, CAPE)], esrc_hbm.at[wid])
    pltpu.sync_copy(dst_v.at[pl.ds(0, CAPE)], edst_hbm.at[wid])

    pltpu.make_async_copy(hs_hbm.at[pl.ds(0, 256), :],
                          sh_hs.at[pl.ds(0, 256), :], sem_st).wait()
    plsc.subcore_barrier()
    _agg_loop(sh_hs, hs_hbm, src_v, dst_v, ec, shared,
              sidx0, didx0, rows0, sem0, sidx1, didx1, rows1, sem1)
    pltpu.sync_copy(shared.at[pl.ds(s * 128, 128), :],
                    agg_hbm.at[pl.ds(wid * 128, 128), :])


def _extract_agg(packed, hs0):
    mesh = plsc.VectorSubcoreMesh(core_axis_name="c", subcore_axis_name="s")
    return pl.kernel(
        _extract_agg_body,
        out_type=[
            jax.ShapeDtypeStruct((NTILES, CAPE), jnp.int32),
            jax.ShapeDtypeStruct((NTILES, CAPE), jnp.int32),
            jax.ShapeDtypeStruct((NTILES * 16,), jnp.int32),
            jax.ShapeDtypeStruct((N, D), jnp.float32),
        ],
        mesh=mesh,
        scratch_types=[
            pltpu.VMEM((GPT, N), jnp.int32),
            pltpu.VMEM((CAPE + 16,), jnp.int32),
            pltpu.VMEM((CAPE + 16,), jnp.int32),
            pltpu.VMEM((CAPW + 16,), jnp.int32),
            pltpu.VMEM((CAPW + 16,), jnp.int32),
            pltpu.VMEM((16,), jnp.int32),
            pltpu.VMEM((CHUNK,), jnp.int32),
            pltpu.VMEM((CHUNK,), jnp.int32),
            pltpu.VMEM((CHUNK, D), jnp.float32),
            pltpu.VMEM((CHUNK,), jnp.int32),
            pltpu.VMEM((CHUNK,), jnp.int32),
            pltpu.VMEM((CHUNK, D), jnp.float32),
            pltpu.VMEM_SHARED((HALF + 16, D), jnp.float32),
            pltpu.VMEM_SHARED((N, D), jnp.float32),
            pltpu.SemaphoreType.DMA,
            pltpu.SemaphoreType.DMA,
            pltpu.SemaphoreType.DMA,
        ],
        compiler_params=pltpu.CompilerParams(needs_layout_passes=False),
    )(packed, hs0)


def _agg2_body(hs_hbm, esrc_hbm, edst_hbm, cnt_hbm, agg_hbm,
               src_v, dst_v, cnt_v,
               sidx0, didx0, rows0, sidx1, didx1, rows1,
               shared, sh_hs, sem0, sem1, sem_st):
    c = lax.axis_index("c")
    s = lax.axis_index("s")
    wid = c * 16 + s

    pltpu.async_copy(hs_hbm.at[pl.ds(s * 256, 256), :],
                     sh_hs.at[pl.ds(s * 256, 256), :], sem_st)
    pltpu.sync_copy(hs_hbm.at[pl.ds(wid * 128, 128), :],
                    shared.at[pl.ds(s * 128, 128), :])
    pltpu.sync_copy(cnt_hbm.at[pl.ds(wid * 16, 16)], cnt_v)
    n = jnp.max(cnt_v[...], axis=0)
    pltpu.sync_copy(esrc_hbm.at[wid], src_v.at[pl.ds(0, CAPE)])
    pltpu.sync_copy(edst_hbm.at[wid], dst_v.at[pl.ds(0, CAPE)])
    pltpu.make_async_copy(hs_hbm.at[pl.ds(0, 256), :],
                          sh_hs.at[pl.ds(0, 256), :], sem_st).wait()
    plsc.subcore_barrier()

    _agg_loop(sh_hs, hs_hbm, src_v, dst_v, n, shared,
              sidx0, didx0, rows0, sem0, sidx1, didx1, rows1, sem1)
    pltpu.sync_copy(shared.at[pl.ds(s * 128, 128), :],
                    agg_hbm.at[pl.ds(wid * 128, 128), :])


def _aggregate2(hs, esrc, edst, cnt):
    mesh = plsc.VectorSubcoreMesh(core_axis_name="c", subcore_axis_name="s")
    return pl.kernel(
        _agg2_body,
        out_type=jax.ShapeDtypeStruct((N, D), jnp.float32),
        mesh=mesh,
        scratch_types=[
            pltpu.VMEM((CAPE + 16,), jnp.int32),
            pltpu.VMEM((CAPE + 16,), jnp.int32),
            pltpu.VMEM((16,), jnp.int32),
            pltpu.VMEM((CHUNK,), jnp.int32),
            pltpu.VMEM((CHUNK,), jnp.int32),
            pltpu.VMEM((CHUNK, D), jnp.float32),
            pltpu.VMEM((CHUNK,), jnp.int32),
            pltpu.VMEM((CHUNK,), jnp.int32),
            pltpu.VMEM((CHUNK, D), jnp.float32),
            pltpu.VMEM_SHARED((HALF + 16, D), jnp.float32),
            pltpu.VMEM_SHARED((N, D), jnp.float32),
            pltpu.SemaphoreType.DMA,
            pltpu.SemaphoreType.DMA,
            pltpu.SemaphoreType.DMA,
        ],
        compiler_params=pltpu.CompilerParams(needs_layout_passes=False),
    )(hs, esrc, edst, cnt)


def _post_body(agg_ref, ni_ref, no_ref, w_ref, b_ref, h_ref, hs_ref):
    hd = agg_ref[...] * ni_ref[...]
    h = jax.nn.relu(
        jnp.dot(hd, w_ref[...], preferred_element_type=jnp.float32)
        + b_ref[...])
    h_ref[...] = h
    hs_ref[...] = h * no_ref[...]


def _post(agg, ni_col, no_col, W, b):
    grid = (4,)
    blk = N // 4
    return pl.pallas_call(
        _post_body,
        grid=grid,
        in_specs=[
            pl.BlockSpec((blk, D), lambda i: (i, 0)),
            pl.BlockSpec((blk, 1), lambda i: (i, 0)),
            pl.BlockSpec((blk, 1), lambda i: (i, 0)),
            pl.BlockSpec((D, D), lambda i: (0, 0)),
            pl.BlockSpec((1, D), lambda i: (0, 0)),
        ],
        out_specs=[
            pl.BlockSpec((blk, D), lambda i: (i, 0)),
            pl.BlockSpec((blk, D), lambda i: (i, 0)),
        ],
        out_shape=[
            jax.ShapeDtypeStruct((N, D), jnp.float32),
            jax.ShapeDtypeStruct((N, D), jnp.float32),
        ],
    )(agg, ni_col, no_col, W, b[None, :])


def kernel(A, features, threshold, W1, b1, W2, b2):
    thr = jnp.asarray(threshold, jnp.float32).reshape(1)
    packed, dout_col, din8, hs0 = _pack_and_degrees(A, features, thr)
    no_col = lax.rsqrt(dout_col + 1.0)
    ni_col = lax.rsqrt(din8[0] + 1.0)[:, None]
    esrc, edst, cnt, agg1 = _extract_agg(packed, hs0)
    h1, hs1 = _post(agg1, ni_col, no_col, W1, b1)
    agg2 = _aggregate2(hs1, esrc, edst, cnt)
    h2, _ = _post(agg2, ni_col, no_col, W2, b2)
    return (h1, h2)


# speedup vs baseline: 2.4528x; 1.0180x over previous
import jax
import jax.numpy as jnp
from jax import lax
from jax.experimental import pallas as pl
from jax.experimental.pallas import tpu as pltpu
from jax.experimental.pallas import tpu_sc as plsc

N = 4096
D = 128
NG32 = N // 32
NTILES = 32
GPT = NG32 // NTILES
CAPW = 4096
CAPE = 4096
CHUNK = 128
HALF = N // 2
DUMMY = HALF

BLK1 = 512


def _pack_body(thr_ref, a_ref, qt_ref, f_ref, pk_ref, dout_ref, din_ref,
               hs_ref):
    i = pl.program_id(0)
    mask_f = (a_ref[...] >= thr_ref[0]).astype(jnp.float32)
    mask = mask_f.astype(jnp.bfloat16)
    pkt = lax.dot_general(mask, qt_ref[...], (((1,), (0,)), ((), ())),
                          preferred_element_type=jnp.float32)
    dout = jnp.sum(mask_f, axis=1, keepdims=True)
    dout_ref[...] = dout
    hs_ref[...] = f_ref[...] * lax.rsqrt(dout + 1.0)

    @pl.when(i == 0)
    def _():
        din_ref[...] = jnp.zeros_like(din_ref)

    ones8 = jnp.full((8, BLK1), jnp.bfloat16(1), jnp.bfloat16)
    din_ref[...] += lax.dot_general(ones8, mask, (((1,), (0,)), ((), ())),
                                    preferred_element_type=jnp.float32)

    pk32 = jnp.transpose(pkt.astype(jnp.int32), (1, 0))
    lo = lax.slice(pk32, (0, 0), (NG32, BLK1))
    hi = lax.slice(pk32, (NG32, 0), (2 * NG32, BLK1))
    pk_ref[...] = lo | lax.shift_left(hi, 16)


def _pack_and_degrees(A, features, thr):
    col = lax.iota(jnp.int32, N)
    gp = lax.iota(jnp.int32, 2 * NG32)[:, None]
    grp16 = jnp.where(gp < NG32, 2 * gp, 2 * (gp - NG32) + 1)
    q = jnp.where((col[None, :] // 16) == grp16,
                  jnp.exp2((col % 16).astype(jnp.float32))[None, :], 0.0)
    qt = q.T.astype(jnp.bfloat16)
    grid = (N // BLK1,)
    return pl.pallas_call(
        _pack_body,
        grid=grid,
        in_specs=[
            pl.BlockSpec(memory_space=pltpu.SMEM),
            pl.BlockSpec((BLK1, N), lambda i: (i, 0)),
            pl.BlockSpec((N, 2 * NG32), lambda i: (0, 0)),
            pl.BlockSpec((BLK1, D), lambda i: (i, 0)),
        ],
        out_specs=[
            pl.BlockSpec((NG32, BLK1), lambda i: (0, i)),
            pl.BlockSpec((BLK1, 1), lambda i: (i, 0)),
            pl.BlockSpec((8, N), lambda i: (0, 0)),
            pl.BlockSpec((BLK1, D), lambda i: (i, 0)),
        ],
        out_shape=[
            jax.ShapeDtypeStruct((NG32, N), jnp.int32),
            jax.ShapeDtypeStruct((N, 1), jnp.float32),
            jax.ShapeDtypeStruct((8, N), jnp.float32),
            jax.ShapeDtypeStruct((N, D), jnp.float32),
        ],
        compiler_params=pltpu.CompilerParams(
            dimension_semantics=("arbitrary",),
        ),
    )(thr, A, qt, features)


def _agg_loop(sh_hs, hs_hbm, src_v, dst_v, n, shared,
              sidx0, didx0, rows0, sem0, sidx1, didx1, rows1, sem1):
    nch = (n + CHUNK - 1) // CHUNK

    def load_idx(ci, sref, dref):
        for k in range(CHUNK // 16):
            sref[pl.ds(k * 16, 16)] = src_v[pl.ds(ci * CHUNK + k * 16, 16)]
            dref[pl.ds(k * 16, 16)] = dst_v[pl.ds(ci * CHUNK + k * 16, 16)]

    def drain(rows, sem):
        pltpu.make_async_copy(hs_hbm.at[pl.ds(0, CHUNK), :], rows, sem).wait()

    @pl.when(nch > 0)
    def _():
        load_idx(0, sidx0, didx0)
        pltpu.async_copy(sh_hs.at[sidx0], rows0, sem0)

    def body(ci, _):
        p = lax.rem(ci, 2)

        @pl.when((p == 0) & (ci + 1 < nch))
        def _():
            @pl.when(ci >= 1)
            def _():
                drain(rows1, sem1)
            load_idx(ci + 1, sidx1, didx1)
            pltpu.async_copy(sh_hs.at[sidx1], rows1, sem1)

        @pl.when((p == 1) & (ci + 1 < nch))
        def _():
            drain(rows0, sem0)
            load_idx(ci + 1, sidx0, didx0)
            pltpu.async_copy(sh_hs.at[sidx0], rows0, sem0)

        @pl.when(p == 0)
        def _():
            drain(rows0, sem0)
            pltpu.async_copy(rows0, shared.at[didx0], sem0, add=True)

        @pl.when(p == 1)
        def _():
            drain(rows1, sem1)
            pltpu.async_copy(rows1, shared.at[didx1], sem1, add=True)

        return 0

    lax.fori_loop(0, nch, body, 0)
    @pl.when(nch > 0)
    def _():
        p = lax.rem(nch - 1, 2)

        @pl.when(p == 0)
        def _():
            drain(rows0, sem0)

        @pl.when(p == 1)
        def _():
            drain(rows1, sem1)

    @pl.when(nch > 1)
    def _():
        p = lax.rem(nch - 2, 2)

        @pl.when(p == 0)
        def _():
            drain(rows0, sem0)

        @pl.when(p == 1)
        def _():
            drain(rows1, sem1)


def _extract_agg_body(pk_hbm, hs_hbm, esrc_hbm, edst_hbm, cnt_hbm, agg_hbm,
                      pk_v, src_v, dst_v, nzw_v, nzi_v, out16_v,
                      sidx0, didx0, rows0, sidx1, didx1, rows1,
                      shared, sh_hs, sem0, sem1, sem_st):
    c = lax.axis_index("c")
    s = lax.axis_index("s")
    wid = c * 16 + s
    g0 = wid * GPT
    iota = lax.iota(jnp.int32, 16)

    pltpu.sync_copy(pk_hbm.at[pl.ds(g0, GPT), :], pk_v)
    pltpu.async_copy(hs_hbm.at[pl.ds(s * 256, 256), :],
                     sh_hs.at[pl.ds(s * 256, 256), :], sem_st)
    pltpu.sync_copy(hs_hbm.at[pl.ds(wid * 128, 128), :],
                    shared.at[pl.ds(s * 128, 128), :])

    offv = jnp.zeros((16,), jnp.int32)
    for gl in range(GPT):
        def body_a(i, offv, gl=gl):
            v = pk_v[gl, pl.ds(i * 16, 16)]
            mm = v != 0
            mi = mm.astype(jnp.int32)
            excl = plsc.cumsum(mi) - mi
            dest = jnp.where(mm, offv + excl, CAPW)
            plsc.store_scatter(nzw_v, [dest], v)
            plsc.store_scatter(nzi_v, [dest], gl * N + i * 16 + iota)
            cnt = plsc.all_reduce_population_count(mm)
            return jnp.minimum(offv + cnt, CAPW - 16)
        offv = lax.fori_loop(0, N // 16, body_a, offv)

    nw = jnp.max(offv, axis=0)
    dbase0 = g0 * 32 - c * HALF

    def body_b(j, ecv):
        lanem = iota < (nw - j * 16)
        w = jnp.where(lanem, nzw_v[pl.ds(j * 16, 16)], 0)
        fi = nzi_v[pl.ds(j * 16, 16)]
        r = fi & (N - 1)
        gl = lax.shift_right_logical(fi, 12)
        dstbase = dbase0 + gl * 32

        def peel_cond(carry):
            return jnp.any(carry[0] != 0)

        def peel(carry):
            w, ecv = carry
            b = w & (-w)
            mm = b != 0
            bf = b.astype(jnp.float32)
            bi = (lax.shift_right_logical(
                lax.bitcast_convert_type(bf, jnp.int32), 23) & 0xFF) - 127
            mi = mm.astype(jnp.int32)
            excl = plsc.cumsum(mi) - mi
            dest = jnp.where(mm, ecv + excl, CAPE)
            plsc.store_scatter(src_v, [dest], r)
            plsc.store_scatter(dst_v, [dest], dstbase + bi)
            cnt = plsc.all_reduce_population_count(mm)
            ecv = jnp.minimum(ecv + cnt, CAPE - 16)
            return (w & (w - 1), ecv)

        _, ecv = lax.while_loop(peel_cond, peel, (w, ecv))
        return ecv

    ecv = lax.fori_loop(0, (nw + 15) // 16, body_b,
                        jnp.zeros((16,), jnp.int32))
    ec = jnp.max(ecv, axis=0)

    target = ((ec + CHUNK - 1) // CHUNK) * CHUNK

    def pad(o):
        src_v[pl.ds(o, 16)] = jnp.zeros((16,), jnp.int32)
        dst_v[pl.ds(o, 16)] = jnp.full((16,), DUMMY, jnp.int32)
        return o + 16

    lax.while_loop(lambda o: o < target, pad, ec)

    out16_v[...] = jnp.full((16,), ec, jnp.int32)
    pltpu.sync_copy(out16_v, cnt_hbm.at[pl.ds(wid * 16, 16)])
    pltpu.sync_copy(src_v.at[pl.ds(0, CAPE)], esrc_hbm.at[wid])
    pltpu.sync_copy(dst_v.at[pl.ds(0, CAPE)], edst_hbm.at[wid])

    pltpu.make_async_copy(hs_hbm.at[pl.ds(0, 256), :],
                          sh_hs.at[pl.ds(0, 256), :], sem_st).wait()
    plsc.subcore_barrier()
    _agg_loop(sh_hs, hs_hbm, src_v, dst_v, ec, shared,
              sidx0, didx0, rows0, sem0, sidx1, didx1, rows1, sem1)
    pltpu.sync_copy(shared.at[pl.ds(s * 128, 128), :],
                    agg_hbm.at[pl.ds(wid * 128, 128), :])


def _extract_agg(packed, hs0):
    mesh = plsc.VectorSubcoreMesh(core_axis_name="c", subcore_axis_name="s")
    return pl.kernel(
        _extract_agg_body,
        out_type=[
            jax.ShapeDtypeStruct((NTILES, CAPE), jnp.int32),
            jax.ShapeDtypeStruct((NTILES, CAPE), jnp.int32),
            jax.ShapeDtypeStruct((NTILES * 16,), jnp.int32),
            jax.ShapeDtypeStruct((N, D), jnp.float32),
        ],
        mesh=mesh,
        scratch_types=[
            pltpu.VMEM((GPT, N), jnp.int32),
            pltpu.VMEM((CAPE + 16,), jnp.int32),
            pltpu.VMEM((CAPE + 16,), jnp.int32),
            pltpu.VMEM((CAPW + 16,), jnp.int32),
            pltpu.VMEM((CAPW + 16,), jnp.int32),
            pltpu.VMEM((16,), jnp.int32),
            pltpu.VMEM((CHUNK,), jnp.int32),
            pltpu.VMEM((CHUNK,), jnp.int32),
            pltpu.VMEM((CHUNK, D), jnp.float32),
            pltpu.VMEM((CHUNK,), jnp.int32),
            pltpu.VMEM((CHUNK,), jnp.int32),
            pltpu.VMEM((CHUNK, D), jnp.float32),
            pltpu.VMEM_SHARED((HALF + 16, D), jnp.float32),
            pltpu.VMEM_SHARED((N, D), jnp.float32),
            pltpu.SemaphoreType.DMA,
            pltpu.SemaphoreType.DMA,
            pltpu.SemaphoreType.DMA,
        ],
        compiler_params=pltpu.CompilerParams(needs_layout_passes=False),
    )(packed, hs0)


def _agg2_body(hs_hbm, esrc_hbm, edst_hbm, cnt_hbm, agg_hbm,
               src_v, dst_v, cnt_v,
               sidx0, didx0, rows0, sidx1, didx1, rows1,
               shared, sh_hs, sem0, sem1, sem_st):
    c = lax.axis_index("c")
    s = lax.axis_index("s")
    wid = c * 16 + s

    pltpu.async_copy(hs_hbm.at[pl.ds(s * 256, 256), :],
                     sh_hs.at[pl.ds(s * 256, 256), :], sem_st)
    pltpu.sync_copy(hs_hbm.at[pl.ds(wid * 128, 128), :],
                    shared.at[pl.ds(s * 128, 128), :])
    pltpu.sync_copy(cnt_hbm.at[pl.ds(wid * 16, 16)], cnt_v)
    n = jnp.max(cnt_v[...], axis=0)
    pltpu.sync_copy(esrc_hbm.at[wid], src_v.at[pl.ds(0, CAPE)])
    pltpu.sync_copy(edst_hbm.at[wid], dst_v.at[pl.ds(0, CAPE)])
    pltpu.make_async_copy(hs_hbm.at[pl.ds(0, 256), :],
                          sh_hs.at[pl.ds(0, 256), :], sem_st).wait()
    plsc.subcore_barrier()

    _agg_loop(sh_hs, hs_hbm, src_v, dst_v, n, shared,
              sidx0, didx0, rows0, sem0, sidx1, didx1, rows1, sem1)
    pltpu.sync_copy(shared.at[pl.ds(s * 128, 128), :],
                    agg_hbm.at[pl.ds(wid * 128, 128), :])


def _aggregate2(hs, esrc, edst, cnt):
    mesh = plsc.VectorSubcoreMesh(core_axis_name="c", subcore_axis_name="s")
    return pl.kernel(
        _agg2_body,
        out_type=jax.ShapeDtypeStruct((N, D), jnp.float32),
        mesh=mesh,
        scratch_types=[
            pltpu.VMEM((CAPE + 16,), jnp.int32),
            pltpu.VMEM((CAPE + 16,), jnp.int32),
            pltpu.VMEM((16,), jnp.int32),
            pltpu.VMEM((CHUNK,), jnp.int32),
            pltpu.VMEM((CHUNK,), jnp.int32),
            pltpu.VMEM((CHUNK, D), jnp.float32),
            pltpu.VMEM((CHUNK,), jnp.int32),
            pltpu.VMEM((CHUNK,), jnp.int32),
            pltpu.VMEM((CHUNK, D), jnp.float32),
            pltpu.VMEM_SHARED((HALF + 16, D), jnp.float32),
            pltpu.VMEM_SHARED((N, D), jnp.float32),
            pltpu.SemaphoreType.DMA,
            pltpu.SemaphoreType.DMA,
            pltpu.SemaphoreType.DMA,
        ],
        compiler_params=pltpu.CompilerParams(needs_layout_passes=False),
    )(hs, esrc, edst, cnt)


def _post_body(agg_ref, ni_ref, no_ref, w_ref, b_ref, h_ref, hs_ref):
    hd = agg_ref[...] * ni_ref[...]
    h = jax.nn.relu(
        jnp.dot(hd, w_ref[...], preferred_element_type=jnp.float32)
        + b_ref[...])
    h_ref[...] = h
    hs_ref[...] = h * no_ref[...]


def _post(agg, ni_col, no_col, W, b):
    grid = (4,)
    blk = N // 4
    return pl.pallas_call(
        _post_body,
        grid=grid,
        in_specs=[
            pl.BlockSpec((blk, D), lambda i: (i, 0)),
            pl.BlockSpec((blk, 1), lambda i: (i, 0)),
            pl.BlockSpec((blk, 1), lambda i: (i, 0)),
            pl.BlockSpec((D, D), lambda i: (0, 0)),
            pl.BlockSpec((1, D), lambda i: (0, 0)),
        ],
        out_specs=[
            pl.BlockSpec((blk, D), lambda i: (i, 0)),
            pl.BlockSpec((blk, D), lambda i: (i, 0)),
        ],
        out_shape=[
            jax.ShapeDtypeStruct((N, D), jnp.float32),
            jax.ShapeDtypeStruct((N, D), jnp.float32),
        ],
    )(agg, ni_col, no_col, W, b[None, :])


def kernel(A, features, threshold, W1, b1, W2, b2):
    thr = jnp.asarray(threshold, jnp.float32).reshape(1)
    packed, dout_col, din8, hs0 = _pack_and_degrees(A, features, thr)
    no_col = lax.rsqrt(dout_col + 1.0)
    ni_col = lax.rsqrt(din8[0] + 1.0)[:, None]
    esrc, edst, cnt, agg1 = _extract_agg(packed, hs0)
    h1, hs1 = _post(agg1, ni_col, no_col, W1, b1)
    agg2 = _aggregate2(hs1, esrc, edst, cnt)
    h2, _ = _post(agg2, ni_col, no_col, W2, b2)
    return (h1, h2)


# speedup vs baseline: 2.5988x; 1.0595x over previous
import jax
import jax.numpy as jnp
from jax import lax
from jax.experimental import pallas as pl
from jax.experimental.pallas import tpu as pltpu
from jax.experimental.pallas import tpu_sc as plsc

N = 4096
D = 128
NG32 = N // 32
NTILES = 32
GPT = NG32 // NTILES
CAPW = 4096
CAPE = 4096
CHUNK = 128
HALF = N // 2
DUMMY = HALF

BLK1 = 512


def _pack_body(thr_ref, a_ref, qt_ref, f_ref, pk_ref, dout_ref, din_ref,
               hs_ref):
    i = pl.program_id(0)
    mask_f = (a_ref[...] >= thr_ref[0]).astype(jnp.float32)
    mask = mask_f.astype(jnp.bfloat16)
    pkt = lax.dot_general(mask, qt_ref[...], (((1,), (0,)), ((), ())),
                          preferred_element_type=jnp.float32)
    dout = jnp.sum(mask_f, axis=1, keepdims=True)
    dout_ref[...] = dout
    hs_ref[...] = f_ref[...] * lax.rsqrt(dout + 1.0)

    @pl.when(i == 0)
    def _():
        din_ref[...] = jnp.zeros_like(din_ref)

    ones8 = jnp.full((8, BLK1), jnp.bfloat16(1), jnp.bfloat16)
    din_ref[...] += lax.dot_general(ones8, mask, (((1,), (0,)), ((), ())),
                                    preferred_element_type=jnp.float32)

    pk32 = jnp.transpose(pkt.astype(jnp.int32), (1, 0))
    lo = lax.slice(pk32, (0, 0), (NG32, BLK1))
    hi = lax.slice(pk32, (NG32, 0), (2 * NG32, BLK1))
    pk_ref[...] = lo | lax.shift_left(hi, 16)


def _pack_and_degrees(A, features, thr):
    col = lax.iota(jnp.int32, N)
    gp = lax.iota(jnp.int32, 2 * NG32)[:, None]
    grp16 = jnp.where(gp < NG32, 2 * gp, 2 * (gp - NG32) + 1)
    q = jnp.where((col[None, :] // 16) == grp16,
                  jnp.exp2((col % 16).astype(jnp.float32))[None, :], 0.0)
    qt = q.T.astype(jnp.bfloat16)
    grid = (N // BLK1,)
    return pl.pallas_call(
        _pack_body,
        grid=grid,
        in_specs=[
            pl.BlockSpec(memory_space=pltpu.SMEM),
            pl.BlockSpec((BLK1, N), lambda i: (i, 0)),
            pl.BlockSpec((N, 2 * NG32), lambda i: (0, 0)),
            pl.BlockSpec((BLK1, D), lambda i: (i, 0)),
        ],
        out_specs=[
            pl.BlockSpec((NG32, BLK1), lambda i: (0, i)),
            pl.BlockSpec((BLK1, 1), lambda i: (i, 0)),
            pl.BlockSpec((8, N), lambda i: (0, 0)),
            pl.BlockSpec((BLK1, D), lambda i: (i, 0)),
        ],
        out_shape=[
            jax.ShapeDtypeStruct((NG32, N), jnp.int32),
            jax.ShapeDtypeStruct((N, 1), jnp.float32),
            jax.ShapeDtypeStruct((8, N), jnp.float32),
            jax.ShapeDtypeStruct((N, D), jnp.float32),
        ],
        compiler_params=pltpu.CompilerParams(
            dimension_semantics=("arbitrary",),
        ),
    )(thr, A, qt, features)


def _agg_loop(sh_hs, hs_hbm, src_v, dst_v, n, shared,
              sidx0, didx0, rows0, sem0, sidx1, didx1, rows1, sem1):
    nch = (n + CHUNK - 1) // CHUNK

    def load_idx(ci, sref, dref):
        for k in range(CHUNK // 16):
            sref[pl.ds(k * 16, 16)] = src_v[pl.ds(ci * CHUNK + k * 16, 16)]
            dref[pl.ds(k * 16, 16)] = dst_v[pl.ds(ci * CHUNK + k * 16, 16)]

    def drain(rows, sem):
        pltpu.make_async_copy(hs_hbm.at[pl.ds(0, CHUNK), :], rows, sem).wait()

    @pl.when(nch > 0)
    def _():
        load_idx(0, sidx0, didx0)
        pltpu.async_copy(sh_hs.at[sidx0], rows0, sem0)

    def body(ci, _):
        p = lax.rem(ci, 2)

        @pl.when((p == 0) & (ci + 1 < nch))
        def _():
            @pl.when(ci >= 1)
            def _():
                drain(rows1, sem1)
            load_idx(ci + 1, sidx1, didx1)
            pltpu.async_copy(sh_hs.at[sidx1], rows1, sem1)

        @pl.when((p == 1) & (ci + 1 < nch))
        def _():
            drain(rows0, sem0)
            load_idx(ci + 1, sidx0, didx0)
            pltpu.async_copy(sh_hs.at[sidx0], rows0, sem0)

        @pl.when(p == 0)
        def _():
            drain(rows0, sem0)
            pltpu.async_copy(rows0, shared.at[didx0], sem0, add=True)

        @pl.when(p == 1)
        def _():
            drain(rows1, sem1)
            pltpu.async_copy(rows1, shared.at[didx1], sem1, add=True)

        return 0

    lax.fori_loop(0, nch, body, 0)
    @pl.when(nch > 0)
    def _():
        p = lax.rem(nch - 1, 2)

        @pl.when(p == 0)
        def _():
            drain(rows0, sem0)

        @pl.when(p == 1)
        def _():
            drain(rows1, sem1)

    @pl.when(nch > 1)
    def _():
        p = lax.rem(nch - 2, 2)

        @pl.when(p == 0)
        def _():
            drain(rows0, sem0)

        @pl.when(p == 1)
        def _():
            drain(rows1, sem1)


def _extract_agg_body(pk_hbm, hs_hbm, esrc_hbm, edst_hbm, cnt_hbm, agg_hbm,
                      pk_v, src_v, dst_v, nzw_v, nzi_v, out16_v,
                      sidx0, didx0, rows0, sidx1, didx1, rows1,
                      shared, sh_hs, sem0, sem1, sem_st):
    c = lax.axis_index("c")
    s = lax.axis_index("s")
    wid = c * 16 + s
    g0 = wid * GPT
    iota = lax.iota(jnp.int32, 16)

    pltpu.sync_copy(pk_hbm.at[pl.ds(g0, GPT), :], pk_v)
    pltpu.async_copy(hs_hbm.at[pl.ds(s * 256, 256), :],
                     sh_hs.at[pl.ds(s * 256, 256), :], sem_st)
    pltpu.sync_copy(hs_hbm.at[pl.ds(wid * 128, 128), :],
                    shared.at[pl.ds(s * 128, 128), :])

    offv = jnp.zeros((16,), jnp.int32)
    for gl in range(GPT):
        def body_a(i, offv, gl=gl):
            v0 = pk_v[gl, pl.ds(i * 32, 16)]
            v1 = pk_v[gl, pl.ds(i * 32 + 16, 16)]
            m0 = v0 != 0
            m1 = v1 != 0
            i0 = m0.astype(jnp.int32)
            i1 = m1.astype(jnp.int32)
            c0 = plsc.all_reduce_population_count(m0)
            c1 = plsc.all_reduce_population_count(m1)
            e0 = plsc.cumsum(i0) - i0
            e1 = plsc.cumsum(i1) - i1
            d0 = jnp.where(m0, offv + e0, CAPW)
            d1 = jnp.where(m1, offv + c0 + e1, CAPW)
            plsc.store_scatter(nzw_v, [d0], v0)
            plsc.store_scatter(nzi_v, [d0], gl * N + i * 32 + iota)
            plsc.store_scatter(nzw_v, [d1], v1)
            plsc.store_scatter(nzi_v, [d1], gl * N + i * 32 + 16 + iota)
            return jnp.minimum(offv + c0 + c1, CAPW - 16)
        offv = lax.fori_loop(0, N // 32, body_a, offv)

    nw = jnp.max(offv, axis=0)
    dbase0 = g0 * 32 - c * HALF

    def body_b(j, ecv):
        lanem = iota < (nw - j * 16)
        w = jnp.where(lanem, nzw_v[pl.ds(j * 16, 16)], 0)
        fi = nzi_v[pl.ds(j * 16, 16)]
        r = fi & (N - 1)
        gl = lax.shift_right_logical(fi, 12)
        dstbase = dbase0 + gl * 32

        def peel_cond(carry):
            return jnp.any(carry[0] != 0)

        def peel(carry):
            w, ecv = carry
            b = w & (-w)
            mm = b != 0
            bf = b.astype(jnp.float32)
            bi = (lax.shift_right_logical(
                lax.bitcast_convert_type(bf, jnp.int32), 23) & 0xFF) - 127
            mi = mm.astype(jnp.int32)
            excl = plsc.cumsum(mi) - mi
            dest = jnp.where(mm, ecv + excl, CAPE)
            plsc.store_scatter(src_v, [dest], r)
            plsc.store_scatter(dst_v, [dest], dstbase + bi)
            cnt = plsc.all_reduce_population_count(mm)
            ecv = jnp.minimum(ecv + cnt, CAPE - 16)
            return (w & (w - 1), ecv)

        _, ecv = lax.while_loop(peel_cond, peel, (w, ecv))
        return ecv

    ecv = lax.fori_loop(0, (nw + 15) // 16, body_b,
                        jnp.zeros((16,), jnp.int32))
    ec = jnp.max(ecv, axis=0)

    target = ((ec + CHUNK - 1) // CHUNK) * CHUNK

    def pad(o):
        src_v[pl.ds(o, 16)] = jnp.zeros((16,), jnp.int32)
        dst_v[pl.ds(o, 16)] = jnp.full((16,), DUMMY, jnp.int32)
        return o + 16

    lax.while_loop(lambda o: o < target, pad, ec)

    out16_v[...] = jnp.full((16,), ec, jnp.int32)
    pltpu.async_copy(out16_v, cnt_hbm.at[pl.ds(wid * 16, 16)], sem_st)
    pltpu.async_copy(src_v.at[pl.ds(0, CAPE)], esrc_hbm.at[wid], sem_st)
    pltpu.async_copy(dst_v.at[pl.ds(0, CAPE)], edst_hbm.at[wid], sem_st)

    pltpu.make_async_copy(hs_hbm.at[pl.ds(0, 256), :],
                          sh_hs.at[pl.ds(0, 256), :], sem_st).wait()
    plsc.subcore_barrier()
    _agg_loop(sh_hs, hs_hbm, src_v, dst_v, ec, shared,
              sidx0, didx0, rows0, sem0, sidx1, didx1, rows1, sem1)
    pltpu.make_async_copy(out16_v, cnt_hbm.at[pl.ds(wid * 16, 16)],
                          sem_st).wait()
    pltpu.make_async_copy(src_v.at[pl.ds(0, CAPE)], esrc_hbm.at[wid],
                          sem_st).wait()
    pltpu.make_async_copy(dst_v.at[pl.ds(0, CAPE)], edst_hbm.at[wid],
                          sem_st).wait()
    pltpu.sync_copy(shared.at[pl.ds(s * 128, 128), :],
                    agg_hbm.at[pl.ds(wid * 128, 128), :])


def _extract_agg(packed, hs0):
    mesh = plsc.VectorSubcoreMesh(core_axis_name="c", subcore_axis_name="s")
    return pl.kernel(
        _extract_agg_body,
        out_type=[
            jax.ShapeDtypeStruct((NTILES, CAPE), jnp.int32),
            jax.ShapeDtypeStruct((NTILES, CAPE), jnp.int32),
            jax.ShapeDtypeStruct((NTILES * 16,), jnp.int32),
            jax.ShapeDtypeStruct((N, D), jnp.float32),
        ],
        mesh=mesh,
        scratch_types=[
            pltpu.VMEM((GPT, N), jnp.int32),
            pltpu.VMEM((CAPE + 16,), jnp.int32),
            pltpu.VMEM((CAPE + 16,), jnp.int32),
            pltpu.VMEM((CAPW + 16,), jnp.int32),
            pltpu.VMEM((CAPW + 16,), jnp.int32),
            pltpu.VMEM((16,), jnp.int32),
            pltpu.VMEM((CHUNK,), jnp.int32),
            pltpu.VMEM((CHUNK,), jnp.int32),
            pltpu.VMEM((CHUNK, D), jnp.float32),
            pltpu.VMEM((CHUNK,), jnp.int32),
            pltpu.VMEM((CHUNK,), jnp.int32),
            pltpu.VMEM((CHUNK, D), jnp.float32),
            pltpu.VMEM_SHARED((HALF + 16, D), jnp.float32),
            pltpu.VMEM_SHARED((N, D), jnp.float32),
            pltpu.SemaphoreType.DMA,
            pltpu.SemaphoreType.DMA,
            pltpu.SemaphoreType.DMA,
        ],
        compiler_params=pltpu.CompilerParams(needs_layout_passes=False),
    )(packed, hs0)


def _agg2_body(hs_hbm, esrc_hbm, edst_hbm, cnt_hbm, agg_hbm,
               src_v, dst_v, cnt_v,
               sidx0, didx0, rows0, sidx1, didx1, rows1,
               shared, sh_hs, sem0, sem1, sem_st):
    c = lax.axis_index("c")
    s = lax.axis_index("s")
    wid = c * 16 + s

    pltpu.async_copy(hs_hbm.at[pl.ds(s * 256, 256), :],
                     sh_hs.at[pl.ds(s * 256, 256), :], sem_st)
    pltpu.sync_copy(hs_hbm.at[pl.ds(wid * 128, 128), :],
                    shared.at[pl.ds(s * 128, 128), :])
    pltpu.sync_copy(cnt_hbm.at[pl.ds(wid * 16, 16)], cnt_v)
    n = jnp.max(cnt_v[...], axis=0)
    pltpu.sync_copy(esrc_hbm.at[wid], src_v.at[pl.ds(0, CAPE)])
    pltpu.sync_copy(edst_hbm.at[wid], dst_v.at[pl.ds(0, CAPE)])
    pltpu.make_async_copy(hs_hbm.at[pl.ds(0, 256), :],
                          sh_hs.at[pl.ds(0, 256), :], sem_st).wait()
    plsc.subcore_barrier()

    _agg_loop(sh_hs, hs_hbm, src_v, dst_v, n, shared,
              sidx0, didx0, rows0, sem0, sidx1, didx1, rows1, sem1)
    pltpu.sync_copy(shared.at[pl.ds(s * 128, 128), :],
                    agg_hbm.at[pl.ds(wid * 128, 128), :])


def _aggregate2(hs, esrc, edst, cnt):
    mesh = plsc.VectorSubcoreMesh(core_axis_name="c", subcore_axis_name="s")
    return pl.kernel(
        _agg2_body,
        out_type=jax.ShapeDtypeStruct((N, D), jnp.float32),
        mesh=mesh,
        scratch_types=[
            pltpu.VMEM((CAPE + 16,), jnp.int32),
            pltpu.VMEM((CAPE + 16,), jnp.int32),
            pltpu.VMEM((16,), jnp.int32),
            pltpu.VMEM((CHUNK,), jnp.int32),
            pltpu.VMEM((CHUNK,), jnp.int32),
            pltpu.VMEM((CHUNK, D), jnp.float32),
            pltpu.VMEM((CHUNK,), jnp.int32),
            pltpu.VMEM((CHUNK,), jnp.int32),
            pltpu.VMEM((CHUNK, D), jnp.float32),
            pltpu.VMEM_SHARED((HALF + 16, D), jnp.float32),
            pltpu.VMEM_SHARED((N, D), jnp.float32),
            pltpu.SemaphoreType.DMA,
            pltpu.SemaphoreType.DMA,
            pltpu.SemaphoreType.DMA,
        ],
        compiler_params=pltpu.CompilerParams(needs_layout_passes=False),
    )(hs, esrc, edst, cnt)


def _post_body(agg_ref, ni_ref, no_ref, w_ref, b_ref, h_ref, hs_ref):
    hd = agg_ref[...] * ni_ref[...]
    h = jax.nn.relu(
        jnp.dot(hd, w_ref[...], preferred_element_type=jnp.float32)
        + b_ref[...])
    h_ref[...] = h
    hs_ref[...] = h * no_ref[...]


def _post(agg, ni_col, no_col, W, b):
    grid = (4,)
    blk = N // 4
    return pl.pallas_call(
        _post_body,
        grid=grid,
        in_specs=[
            pl.BlockSpec((blk, D), lambda i: (i, 0)),
            pl.BlockSpec((blk, 1), lambda i: (i, 0)),
            pl.BlockSpec((blk, 1), lambda i: (i, 0)),
            pl.BlockSpec((D, D), lambda i: (0, 0)),
            pl.BlockSpec((1, D), lambda i: (0, 0)),
        ],
        out_specs=[
            pl.BlockSpec((blk, D), lambda i: (i, 0)),
            pl.BlockSpec((blk, D), lambda i: (i, 0)),
        ],
        out_shape=[
            jax.ShapeDtypeStruct((N, D), jnp.float32),
            jax.ShapeDtypeStruct((N, D), jnp.float32),
        ],
    )(agg, ni_col, no_col, W, b[None, :])


def kernel(A, features, threshold, W1, b1, W2, b2):
    thr = jnp.asarray(threshold, jnp.float32).reshape(1)
    packed, dout_col, din8, hs0 = _pack_and_degrees(A, features, thr)
    no_col = lax.rsqrt(dout_col + 1.0)
    ni_col = lax.rsqrt(din8[0] + 1.0)[:, None]
    esrc, edst, cnt, agg1 = _extract_agg(packed, hs0)
    h1, hs1 = _post(agg1, ni_col, no_col, W1, b1)
    agg2 = _aggregate2(hs1, esrc, edst, cnt)
    h2, _ = _post(agg2, ni_col, no_col, W2, b2)
    return (h1, h2)
